# Initial kernel scaffold; baseline (speedup 1.0000x reference)
import jax
import jax.numpy as jnp
from jax.experimental import pallas as pl


def kernel(atom_type, pos, bond_index, bond_type, batch, graph_idx, time_step, template_mask, edge_index_a, params):
    raise NotImplementedError("write your pallas kernel here")


# speedup vs baseline: 1.5016x; 1.5016x over previous
import functools
import math

import jax
import jax.numpy as jnp
from jax import lax
from jax.experimental import pallas as pl
from jax.experimental.pallas import tpu as pltpu
from jax.experimental.pallas import tpu_sc as plsc

N_NODES = 10000
N_EDGES = 160000
HID = 128

NC = 2
NS = 16
NW = NC * NS
CHUNK = 128

NPAD = 10240
DUMMY = N_NODES
EPAD = 163840
E_PER_TILE = EPAD // NW
CH_PER_TILE = E_PER_TILE // CHUNK
ROWS_PER_TILE = NPAD // NS
EXPORT_CH = ROWS_PER_TILE // CHUNK

BE = 1024
BN = 1024

_mesh = functools.partial(
    plsc.VectorSubcoreMesh, core_axis_name="c", subcore_axis_name="s",
    num_cores=NC, num_subcores=NS)


def _gather_body(n_tasks, widths, *refs):
  tabs = refs[0:n_tasks]
  idxs = refs[n_tasks:2 * n_tasks]
  outs = refs[2 * n_tasks:3 * n_tasks]
  scr = refs[3 * n_tasks:]
  idx_bufs = scr[0:n_tasks]
  row_bufs = scr[n_tasks:2 * n_tasks]
  sem = scr[2 * n_tasks]
  cid = lax.axis_index("c")
  sid = lax.axis_index("s")
  wid = sid * NC + cid
  base = wid * E_PER_TILE

  def chunk(c, carry):
    eb = base + c * CHUNK
    for k in range(n_tasks):
      pltpu.sync_copy(idxs[k].at[pl.ds(eb, CHUNK)], idx_bufs[k])
      pltpu.async_copy(tabs[k].at[idx_bufs[k]], row_bufs[k], sem).wait()
      pltpu.sync_copy(row_bufs[k], outs[k].at[pl.ds(eb, CHUNK)])
    return carry

  lax.fori_loop(0, CH_PER_TILE, chunk, 0)


def _sc_gather(tables, indices):
  n = len(tables)
  widths = [t.shape[1] for t in tables]
  out_type = [jax.ShapeDtypeStruct((EPAD, w), jnp.float32) for w in widths]
  scratch = ([pltpu.VMEM((CHUNK,), jnp.int32) for _ in range(n)]
             + [pltpu.VMEM((CHUNK, w), jnp.float32) for w in widths]
             + [pltpu.SemaphoreType.DMA])
  fn = pl.kernel(
      functools.partial(_gather_body, n, tuple(widths)),
      out_type=out_type, mesh=_mesh(), scratch_types=scratch)
  return fn(*tables, *indices)


def _scatter_body(mm, row, out_h, acc_h, buf_m, idx_b):
  cid = lax.axis_index("c")
  sid = lax.axis_index("s")
  wid = sid * NC + cid
  base = wid * E_PER_TILE

  zero = jnp.zeros((16,), jnp.float32)

  def zinit(i, carry):
    for j in range(HID // 16):
      buf_m[i, pl.ds(j * 16, 16)] = zero
    return carry
  lax.fori_loop(0, CHUNK, zinit, 0)

  def zfill(k, carry):
    r = sid * ROWS_PER_TILE + k * CHUNK
    pltpu.sync_copy(buf_m, acc_h.at[pl.ds(r, CHUNK)])
    return carry
  lax.fori_loop(0, EXPORT_CH, zfill, 0)
  plsc.subcore_barrier()

  def chunk(c, carry):
    eb = base + c * CHUNK
    pltpu.sync_copy(mm.at[pl.ds(eb, CHUNK)], buf_m)
    pltpu.sync_copy(row.at[pl.ds(eb, CHUNK)], idx_b)
    pltpu.sync_copy(buf_m, acc_h.at[idx_b], add=True)
    return carry
  lax.fori_loop(0, CH_PER_TILE, chunk, 0)
  plsc.subcore_barrier()

  def export(k, carry):
    r = sid * ROWS_PER_TILE + k * CHUNK
    pltpu.sync_copy(acc_h.at[pl.ds(r, CHUNK)], buf_m)
    pltpu.sync_copy(buf_m, out_h.at[pl.ds(cid * NPAD + r, CHUNK)])
    return carry
  lax.fori_loop(0, EXPORT_CH, export, 0)


def _sc_scatter(mm, row):
  out_type = jax.ShapeDtypeStruct((NC * NPAD, HID), jnp.float32)
  scratch = [
      pltpu.VMEM_SHARED((NPAD, HID), jnp.float32),
      pltpu.VMEM((CHUNK, HID), jnp.float32),
      pltpu.VMEM((CHUNK,), jnp.int32),
  ]
  fn = pl.kernel(_scatter_body, out_type=out_type, mesh=_mesh(),
                 scratch_types=scratch)
  return fn(mm, row)


def _silu(v):
  return v * jax.nn.sigmoid(v)


def _rep(shape):
  nd = len(shape)
  return pl.BlockSpec(shape, lambda i: (0,) * nd)


def _node_pre_body(h, xa, wa, wb, ta, tb):
  hv = h[...]
  xav = xa[...]
  z = jnp.zeros((BN, 112), jnp.float32)
  ta[...] = jnp.concatenate(
      [jnp.dot(hv, wa[...], preferred_element_type=jnp.float32), xav, z], axis=1)
  tb[...] = jnp.concatenate(
      [jnp.dot(hv, wb[...], preferred_element_type=jnp.float32), xav, z], axis=1)


def _node_pre(h, xa, wa, wb):
  grid = NPAD // BN
  return pl.pallas_call(
      _node_pre_body,
      grid=(grid,),
      in_specs=[pl.BlockSpec((BN, HID), lambda i: (i, 0)),
                pl.BlockSpec((BN, 16), lambda i: (i, 0)),
                _rep((HID, HID)), _rep((HID, HID))],
      out_specs=[pl.BlockSpec((BN, 256), lambda i: (i, 0))] * 2,
      out_shape=[jax.ShapeDtypeStruct((NPAD, 256), jnp.float32)] * 2,
  )(h, xa, wa, wb)


def _edge_mlp_body(g1x, g2x, eat, wea, b1, wd, we2, b2,
                   watt, batt, wx1, bx1, wx2, bx2, mm_o, pay_o):
  lane = lax.broadcasted_iota(jnp.int32, (BE, 16), 1)
  g1v = g1x[...]
  g2v = g2x[...]
  diff = jnp.where(lane < 3, g1v[:, HID:HID + 16] - g2v[:, HID:HID + 16], 0.0)
  d2 = jnp.sum(diff * diff, axis=1, keepdims=True)
  pre = (g1v[:, 0:HID] + g2v[:, 0:HID] + d2 * wd[...]
         + jnp.dot(eat[...], wea[...], preferred_element_type=jnp.float32)
         + b1[...])
  m = _silu(pre)
  m = _silu(jnp.dot(m, we2[...], preferred_element_type=jnp.float32) + b2[...])
  att = jax.nn.sigmoid(
      jnp.dot(m, watt[...], preferred_element_type=jnp.float32) + batt[...])[:, 0:1]
  mm = m * att
  t = _silu(jnp.dot(mm, wx1[...], preferred_element_type=jnp.float32) + bx1[...])
  tx = (jnp.dot(t, wx2[...], preferred_element_type=jnp.float32) + bx2[...])[:, 0:1]
  mm_o[...] = mm
  pay16 = diff * tx + (lane == 3).astype(jnp.float32)
  pay_o[...] = jnp.concatenate([pay16, jnp.zeros((BE, 112), jnp.float32)], axis=1)


def _edge_mlp(g1x, g2x, eat, w):
  ew = eat.shape[1]
  grid = EPAD // BE
  eb = pl.BlockSpec((BE, HID), lambda i: (i, 0))
  e256 = pl.BlockSpec((BE, 256), lambda i: (i, 0))
  e16 = pl.BlockSpec((BE, 16), lambda i: (i, 0))
  return pl.pallas_call(
      _edge_mlp_body,
      grid=(grid,),
      in_specs=[e256, e256, pl.BlockSpec((BE, ew), lambda i: (i, 0)),
                _rep((ew, HID)), _rep((1, HID)), _rep((1, HID)),
                _rep((HID, HID)), _rep((1, HID)),
                _rep((HID, 8)), _rep((1, 8)),
                _rep((HID, HID)), _rep((1, HID)),
                _rep((HID, 8)), _rep((1, 8))],
      out_specs=[eb, eb],
      out_shape=[jax.ShapeDtypeStruct((EPAD, HID), jnp.float32),
                 jax.ShapeDtypeStruct((EPAD, HID), jnp.float32)],
  )(g1x, g2x, eat, w["wea"], w["b1"], w["wd"], w["we2"], w["b2"],
    w["watt"], w["batt"], w["wx1"], w["bx1"], w["wx2"], w["bx2"])


def _node_upd_body(h, xa, p0, p1, q0, q1, wh1a, wh1b, bh1, wh2, bh2,
                   h_o, x_o):
  hv = h[...]
  agg = p0[...] + p1[...]
  u = _silu(jnp.dot(hv, wh1a[...], preferred_element_type=jnp.float32)
            + jnp.dot(agg, wh1b[...], preferred_element_type=jnp.float32)
            + bh1[...])
  h_o[...] = hv + jnp.dot(u, wh2[...], preferred_element_type=jnp.float32) + bh2[...]
  q = q0[...] + q1[...]
  cnt = q[:, 3:4]
  xv = xa[...]
  mask = xv[:, 4:5]
  upd = q[:, 0:16] * (mask / jnp.maximum(cnt, 1.0))
  lane = lax.broadcasted_iota(jnp.int32, (BN, 16), 1)
  x_o[...] = xv + jnp.where(lane < 3, upd, 0.0)


def _node_upd(h, xa, p0, p1, q0, q1, w):
  grid = NPAD // BN
  nb = pl.BlockSpec((BN, HID), lambda i: (i, 0))
  n16 = pl.BlockSpec((BN, 16), lambda i: (i, 0))
  return pl.pallas_call(
      _node_upd_body,
      grid=(grid,),
      in_specs=[nb, n16, nb, nb, nb, nb,
                _rep((HID, HID)), _rep((HID, HID)), _rep((1, HID)),
                _rep((HID, HID)), _rep((1, HID))],
      out_specs=[nb, n16],
      out_shape=[jax.ShapeDtypeStruct((NPAD, HID), jnp.float32),
                 jax.ShapeDtypeStruct((NPAD, 16), jnp.float32)],
  )(h, xa, p0, p1, q0, q1, w["wh1a"], w["wh1b"], w["bh1"], w["wh2"], w["bh2"])


def _edge_enc_body(g1x, g2x, emb, m1w, m1b, m2w, m2b, e_o):
  lane = lax.broadcasted_iota(jnp.int32, (BE, 16), 1)
  diff = jnp.where(lane < 3,
                   g1x[:, HID:HID + 16] - g2x[:, HID:HID + 16], 0.0)
  el = jnp.sqrt(jnp.sum(diff * diff, axis=1, keepdims=True) + 1e-12)
  d = jax.nn.relu(el * m1w[...] + m1b[...])
  d = jnp.dot(d, m2w[...], preferred_element_type=jnp.float32) + m2b[...]
  e_o[...] = d * emb[...]


def _edge_enc(g1x, g2x, emb, m1w, m1b, m2w, m2b):
  grid = EPAD // BE
  eb = pl.BlockSpec((BE, HID), lambda i: (i, 0))
  e256 = pl.BlockSpec((BE, 256), lambda i: (i, 0))
  return pl.pallas_call(
      _edge_enc_body,
      grid=(grid,),
      in_specs=[e256, e256, eb, _rep((1, HID)), _rep((1, HID)),
                _rep((HID, HID)), _rep((1, HID))],
      out_specs=eb,
      out_shape=jax.ShapeDtypeStruct((EPAD, HID), jnp.float32),
  )(g1x, g2x, emb, m1w, m1b, m2w, m2b)


def _edge_len_body(g1x, g2x, e_o):
  lane = lax.broadcasted_iota(jnp.int32, (BE, 16), 1)
  diff = jnp.where(lane < 3,
                   g1x[:, HID:HID + 16] - g2x[:, HID:HID + 16], 0.0)
  el = jnp.sqrt(jnp.sum(diff * diff, axis=1, keepdims=True) + 1e-12)
  e_o[...] = jnp.where(lane == 0, el, 0.0)


def _edge_len(g1x, g2x):
  grid = EPAD // BE
  e256 = pl.BlockSpec((BE, 256), lambda i: (i, 0))
  e16 = pl.BlockSpec((BE, 16), lambda i: (i, 0))
  return pl.pallas_call(
      _edge_len_body, grid=(grid,), in_specs=[e256, e256], out_specs=e16,
      out_shape=jax.ShapeDtypeStruct((EPAD, 16), jnp.float32),
  )(g1x, g2x)


def _init_body(ni, nemb, wnl, bnl, tsf, freqs, wt1, bt1, wt2, bt2,
               temb_tab, wtl, btl, h_o):
  ai = ni[:, 0:1]
  oh_a = (lax.broadcasted_iota(jnp.int32, (BN, 128), 1) == ai).astype(jnp.float32)
  sil_tab = _silu(nemb[...])
  nv = jnp.dot(jnp.dot(oh_a, sil_tab, preferred_element_type=jnp.float32),
               wnl[...], preferred_element_type=jnp.float32) + bnl[...]
  bi = ni[:, 1:2]
  oh_b = (lax.broadcasted_iota(jnp.int32, (BN, 64), 1) == bi).astype(jnp.float32)
  t0 = jnp.dot(oh_b, tsf[...], preferred_element_type=jnp.float32)[:, 0:1]
  ang = t0 * freqs[...]
  te = jnp.concatenate([jnp.sin(ang), jnp.cos(ang)], axis=1)
  te = _silu(jnp.dot(te, wt1[...], preferred_element_type=jnp.float32) + bt1[...])
  te = jnp.dot(te, wt2[...], preferred_element_type=jnp.float32) + bt2[...]
  mrows = jnp.dot(_silu(temb_tab[...]), wtl[...],
                  preferred_element_type=jnp.float32) + btl[...]
  tif = (ni[:, 2:3] == 1).astype(jnp.float32)
  memb = mrows[0:1, :] + tif * (mrows[1:2, :] - mrows[0:1, :])
  h_o[...] = jnp.concatenate([nv, te, memb], axis=1)


def _init_h(ni, p):
  grid = NPAD // BN
  return pl.pallas_call(
      _init_body,
      grid=(grid,),
      in_specs=[pl.BlockSpec((BN, 8), lambda i: (i, 0)),
                _rep((128, 64)), _rep((64, 64)), _rep((1, 64)),
                _rep((64, 8)), _rep((1, 16)),
                _rep((32, 32)), _rep((1, 32)), _rep((32, 32)), _rep((1, 32)),
                _rep((8, 32)), _rep((32, 32)), _rep((1, 32))],
      out_specs=pl.BlockSpec((BN, HID), lambda i: (i, 0)),
      out_shape=jax.ShapeDtypeStruct((NPAD, HID), jnp.float32),
  )(ni, p["nemb"], p["wnl"], p["bnl"], p["tsf"], p["freqs"],
    p["wt1"], p["bt1"], p["wt2"], p["bt2"], p["temb_tab"], p["wtl"], p["btl"])


def _prep_egcl(p):
  we1 = p["e1"]["w"]
  enf = we1.shape[0] - 2 * HID - 1
  wea = we1[2 * HID + 1:]
  if enf == 1:
    wea = jnp.pad(wea, ((0, 15), (0, 0)))
  return {
      "wa": we1[0:HID], "wb": we1[HID:2 * HID],
      "wd": we1[2 * HID:2 * HID + 1], "wea": wea,
      "b1": p["e1"]["b"][None, :],
      "we2": p["e2"]["w"], "b2": p["e2"]["b"][None, :],
      "watt": jnp.pad(p["att"]["w"], ((0, 0), (0, 7))),
      "batt": jnp.pad(p["att"]["b"][None, :], ((0, 0), (0, 7))),
      "wx1": p["x1"]["w"], "bx1": p["x1"]["b"][None, :],
      "wx2": jnp.pad(p["x2"]["w"], ((0, 0), (0, 7))),
      "bx2": jnp.pad(p["x2"]["b"][None, :], ((0, 0), (0, 7))),
      "wh1a": p["h1"]["w"][0:HID], "wh1b": p["h1"]["w"][HID:],
      "bh1": p["h1"]["b"][None, :],
      "wh2": p["h2"]["w"], "bh2": p["h2"]["b"][None, :],
  }


def _pad_edges(idx, fill):
  return jnp.concatenate(
      [idx.astype(jnp.int32),
       jnp.full((EPAD - N_EDGES,), fill, jnp.int32)])


def kernel(atom_type, pos, bond_index, bond_type, batch, graph_idx,
           time_step, template_mask, edge_index_a, params):
  del graph_idx
  row_b = _pad_edges(bond_index[0], DUMMY)
  col_b = _pad_edges(bond_index[1], 0)
  row_a = _pad_edges(edge_index_a[0], DUMMY)
  col_a = _pad_edges(edge_index_a[1], 0)
  typ = _pad_edges(bond_type, 0)

  maskf = template_mask.astype(jnp.float32)
  xa = jnp.zeros((NPAD, 16), jnp.float32)
  xa = xa.at[:N_NODES, 0:3].set(pos)
  xa = xa.at[:N_NODES, 4].set(maskf)

  ni = jnp.zeros((NPAD, 8), jnp.int32)
  ni = ni.at[:N_NODES, 0].set(atom_type.astype(jnp.int32))
  ni = ni.at[:N_NODES, 1].set(batch.astype(jnp.int32))
  ni = ni.at[:N_NODES, 2].set(template_mask.astype(jnp.int32))

  half = 16
  scale = math.log(10000.0) / (half - 1)
  freqs = jnp.exp(jnp.arange(half, dtype=jnp.float32) * -scale)[None, :]
  tsf = jnp.zeros((64, 8), jnp.float32).at[:, 0].set(
      time_step.astype(jnp.float32))

  ip = {
      "nemb": jnp.pad(params["node_emb"], ((0, 28), (0, 0))),
      "wnl": params["node_lin"]["w"], "bnl": params["node_lin"]["b"][None, :],
      "tsf": tsf, "freqs": freqs,
      "wt1": params["t1"]["w"], "bt1": params["t1"]["b"][None, :],
      "wt2": params["t2"]["w"], "bt2": params["t2"]["b"][None, :],
      "temb_tab": jnp.pad(params["tmpl_emb"], ((0, 6), (0, 0))),
      "wtl": params["tmpl_lin"]["w"], "btl": params["tmpl_lin"]["b"][None, :],
  }

  enc1 = [_prep_egcl(p) for p in params["enc1"]]
  encc = [_prep_egcl(p) for p in params["enc_cross"]]
  enc2 = [_prep_egcl(p) for p in params["enc2"]]

  ee1 = params["edge_enc"]
  ee2 = params["edge_enc2"]

  h = _init_h(ni, ip)

  emb1_pad = jnp.pad(ee1["emb"], ((0, 28), (0, 0)))
  emb2_pad = jnp.pad(ee2["emb"], ((0, 28), (0, 0)))
  emb_rows1, emb_rows2 = _sc_gather([emb1_pad, emb2_pad], [typ, typ])

  def gather_layer(h, xa, w, row, col):
    ta, tb = _node_pre(h, xa, w["wa"], w["wb"])
    return _sc_gather([ta, tb], [row, col])

  def finish_layer(h, xa, w, row, g1x, g2x, eat):
    mm, pay = _edge_mlp(g1x, g2x, eat, w)
    agg_h = _sc_scatter(mm, row)
    agg_x = _sc_scatter(pay, row)
    return _node_upd(h, xa, agg_h[:NPAD], agg_h[NPAD:],
                     agg_x[:NPAD], agg_x[NPAD:], w)

  e1buf = None
  for li, w in enumerate(enc1):
    g1x, g2x = gather_layer(h, xa, w, row_b, col_b)
    if li == 0:
      e1buf = _edge_enc(g1x, g2x, emb_rows1, ee1["m1"]["w"],
                        ee1["m1"]["b"][None, :], ee1["m2"]["w"],
                        ee1["m2"]["b"][None, :])
    h, xa = finish_layer(h, xa, w, row_b, g1x, g2x, e1buf)

  ela = None
  for li, w in enumerate(encc):
    g1x, g2x = gather_layer(h, xa, w, row_a, col_a)
    if li == 0:
      ela = _edge_len(g1x, g2x)
    h, xa = finish_layer(h, xa, w, row_a, g1x, g2x, ela)

  e2buf = None
  for li, w in enumerate(enc2):
    g1x, g2x = gather_layer(h, xa, w, row_b, col_b)
    if li == 0:
      e2buf = _edge_enc(g1x, g2x, emb_rows2, ee2["m1"]["w"],
                        ee2["m1"]["b"][None, :], ee2["m2"]["w"],
                        ee2["m2"]["b"][None, :])
    h, xa = finish_layer(h, xa, w, row_b, g1x, g2x, e2buf)

  return xa[:N_NODES, 0:3] - pos


# speedup vs baseline: 1.7758x; 1.1826x over previous
import functools
import math

import jax
import jax.numpy as jnp
from jax import lax
from jax.experimental import pallas as pl
from jax.experimental.pallas import tpu as pltpu
from jax.experimental.pallas import tpu_sc as plsc

N_NODES = 10000
N_EDGES = 160000
HID = 128

NC = 2
NS = 16
NW = NC * NS
CHUNK = 128

NPAD = 10240
DUMMY = N_NODES
EPAD = 163840
E_PER_TILE = EPAD // NW
CH_PER_TILE = E_PER_TILE // CHUNK
ROWS_PER_TILE = NPAD // NS
EXPORT_CH = ROWS_PER_TILE // CHUNK

BE = 1024
BN = 1024

_mesh = functools.partial(
    plsc.VectorSubcoreMesh, core_axis_name="c", subcore_axis_name="s",
    num_cores=NC, num_subcores=NS)


def _gather_body(n_tasks, widths, *refs):
  tabs = refs[0:n_tasks]
  idxs = refs[n_tasks:2 * n_tasks]
  outs = refs[2 * n_tasks:3 * n_tasks]
  scr = refs[3 * n_tasks:]
  idx_bufs = scr[0:n_tasks]
  row_bufs = scr[n_tasks:2 * n_tasks]
  sems = scr[2 * n_tasks:3 * n_tasks]
  cid = lax.axis_index("c")
  sid = lax.axis_index("s")
  wid = sid * NC + cid
  base = wid * E_PER_TILE

  def chunk(c, carry):
    eb = base + c * CHUNK
    d = [pltpu.async_copy(idxs[k].at[pl.ds(eb, CHUNK)], idx_bufs[k], sems[k])
         for k in range(n_tasks)]
    for x in d:
      x.wait()
    d = [pltpu.async_copy(tabs[k].at[idx_bufs[k]], row_bufs[k], sems[k])
         for k in range(n_tasks)]
    for x in d:
      x.wait()
    d = [pltpu.async_copy(row_bufs[k], outs[k].at[pl.ds(eb, CHUNK)], sems[k])
         for k in range(n_tasks)]
    for x in d:
      x.wait()
    return carry

  lax.fori_loop(0, CH_PER_TILE, chunk, 0)


def _sc_gather(tables, indices):
  n = len(tables)
  widths = [t.shape[1] for t in tables]
  out_type = [jax.ShapeDtypeStruct((EPAD, w), jnp.float32) for w in widths]
  scratch = ([pltpu.VMEM((CHUNK,), jnp.int32) for _ in range(n)]
             + [pltpu.VMEM((CHUNK, w), jnp.float32) for w in widths]
             + [pltpu.SemaphoreType.DMA for _ in range(n)])
  fn = pl.kernel(
      functools.partial(_gather_body, n, tuple(widths)),
      out_type=out_type, mesh=_mesh(), scratch_types=scratch)
  return fn(*tables, *indices)


def _scatter_body(mm, pay, row, out_h, out_x, acc, buf_m, buf_p, idx_b):
  cid = lax.axis_index("c")
  sid = lax.axis_index("s")
  wid = sid * NC + cid
  base = wid * E_PER_TILE

  zero = jnp.zeros((16,), jnp.float32)

  def zinit(i, carry):
    for j in range(HID // 16):
      buf_m[i, pl.ds(j * 16, 16)] = zero
    return carry

  def zfill(k, carry):
    r = sid * ROWS_PER_TILE + k * CHUNK
    pltpu.sync_copy(buf_m, acc.at[pl.ds(r, CHUNK)])
    return carry

  def export(out):
    def body(k, carry):
      r = sid * ROWS_PER_TILE + k * CHUNK
      pltpu.sync_copy(acc.at[pl.ds(r, CHUNK)], buf_m)
      pltpu.sync_copy(buf_m, out.at[pl.ds(cid * NPAD + r, CHUNK)])
      return carry
    return body

  lax.fori_loop(0, CHUNK, zinit, 0)
  lax.fori_loop(0, EXPORT_CH, zfill, 0)
  plsc.subcore_barrier()

  def chunk_mm(c, carry):
    eb = base + c * CHUNK
    pltpu.sync_copy(mm.at[pl.ds(eb, CHUNK)], buf_m)
    pltpu.sync_copy(row.at[pl.ds(eb, CHUNK)], idx_b)
    pltpu.sync_copy(buf_m, acc.at[idx_b], add=True)
    return carry
  lax.fori_loop(0, CH_PER_TILE, chunk_mm, 0)
  plsc.subcore_barrier()
  lax.fori_loop(0, EXPORT_CH, export(out_h), 0)
  plsc.subcore_barrier()

  lax.fori_loop(0, CHUNK, zinit, 0)
  lax.fori_loop(0, EXPORT_CH, zfill, 0)
  plsc.subcore_barrier()
  lax.fori_loop(0, CHUNK, zinit, 0)

  def chunk_pay(c, carry):
    eb = base + c * CHUNK
    pltpu.sync_copy(pay.at[pl.ds(eb, CHUNK)], buf_p)
    pltpu.sync_copy(row.at[pl.ds(eb, CHUNK)], idx_b)

    def fill(i, carry2):
      buf_m[i, pl.ds(0, 16)] = buf_p[i]
      return carry2
    lax.fori_loop(0, CHUNK, fill, 0)
    pltpu.sync_copy(buf_m, acc.at[idx_b], add=True)
    return carry
  lax.fori_loop(0, CH_PER_TILE, chunk_pay, 0)
  plsc.subcore_barrier()
  lax.fori_loop(0, EXPORT_CH, export(out_x), 0)


def _sc_scatter(mm, pay, row):
  out_type = [jax.ShapeDtypeStruct((NC * NPAD, HID), jnp.float32),
              jax.ShapeDtypeStruct((NC * NPAD, HID), jnp.float32)]
  scratch = [
      pltpu.VMEM_SHARED((NPAD, HID), jnp.float32),
      pltpu.VMEM((CHUNK, HID), jnp.float32),
      pltpu.VMEM((CHUNK, 16), jnp.float32),
      pltpu.VMEM((CHUNK,), jnp.int32),
  ]
  fn = pl.kernel(_scatter_body, out_type=out_type, mesh=_mesh(),
                 scratch_types=scratch)
  return fn(mm, pay, row)


def _silu(v):
  return v * jax.nn.sigmoid(v)


def _rep(shape):
  nd = len(shape)
  return pl.BlockSpec(shape, lambda i: (0,) * nd)


def _node_pre_body(h, xa, wa, wb, ta, tb):
  hv = h[...]
  xav = xa[...]
  z = jnp.zeros((BN, 112), jnp.float32)
  ta[...] = jnp.concatenate(
      [jnp.dot(hv, wa[...], preferred_element_type=jnp.float32), xav, z], axis=1)
  tb[...] = jnp.concatenate(
      [jnp.dot(hv, wb[...], preferred_element_type=jnp.float32), xav, z], axis=1)


def _node_pre(h, xa, wa, wb):
  grid = NPAD // BN
  return pl.pallas_call(
      _node_pre_body,
      grid=(grid,),
      in_specs=[pl.BlockSpec((BN, HID), lambda i: (i, 0)),
                pl.BlockSpec((BN, 16), lambda i: (i, 0)),
                _rep((HID, HID)), _rep((HID, HID))],
      out_specs=[pl.BlockSpec((BN, 256), lambda i: (i, 0))] * 2,
      out_shape=[jax.ShapeDtypeStruct((NPAD, 256), jnp.float32)] * 2,
  )(h, xa, wa, wb)


def _edge_mlp_body(g1x, g2x, eat, wea, b1, wd, we2, b2,
                   watt, batt, wx1, bx1, wx2, bx2, mm_o, pay_o):
  lane = lax.broadcasted_iota(jnp.int32, (BE, 16), 1)
  g1v = g1x[...]
  g2v = g2x[...]
  diff = jnp.where(lane < 3, g1v[:, HID:HID + 16] - g2v[:, HID:HID + 16], 0.0)
  d2 = jnp.sum(diff * diff, axis=1, keepdims=True)
  pre = (g1v[:, 0:HID] + g2v[:, 0:HID] + d2 * wd[...]
         + jnp.dot(eat[...], wea[...], preferred_element_type=jnp.float32)
         + b1[...])
  m = _silu(pre)
  m = _silu(jnp.dot(m, we2[...], preferred_element_type=jnp.float32) + b2[...])
  att = jax.nn.sigmoid(
      jnp.dot(m, watt[...], preferred_element_type=jnp.float32) + batt[...])[:, 0:1]
  mm = m * att
  t = _silu(jnp.dot(mm, wx1[...], preferred_element_type=jnp.float32) + bx1[...])
  tx = (jnp.dot(t, wx2[...], preferred_element_type=jnp.float32) + bx2[...])[:, 0:1]
  mm_o[...] = mm
  pay_o[...] = diff * tx + (lane == 3).astype(jnp.float32)


def _edge_mlp(g1x, g2x, eat, w):
  ew = eat.shape[1]
  grid = EPAD // BE
  eb = pl.BlockSpec((BE, HID), lambda i: (i, 0))
  e256 = pl.BlockSpec((BE, 256), lambda i: (i, 0))
  e16 = pl.BlockSpec((BE, 16), lambda i: (i, 0))
  return pl.pallas_call(
      _edge_mlp_body,
      grid=(grid,),
      in_specs=[e256, e256, pl.BlockSpec((BE, ew), lambda i: (i, 0)),
                _rep((ew, HID)), _rep((1, HID)), _rep((1, HID)),
                _rep((HID, HID)), _rep((1, HID)),
                _rep((HID, 8)), _rep((1, 8)),
                _rep((HID, HID)), _rep((1, HID)),
                _rep((HID, 8)), _rep((1, 8))],
      out_specs=[eb, e16],
      out_shape=[jax.ShapeDtypeStruct((EPAD, HID), jnp.float32),
                 jax.ShapeDtypeStruct((EPAD, 16), jnp.float32)],
  )(g1x, g2x, eat, w["wea"], w["b1"], w["wd"], w["we2"], w["b2"],
    w["watt"], w["batt"], w["wx1"], w["bx1"], w["wx2"], w["bx2"])


def _node_upd_body(h, xa, p0, p1, q0, q1, wh1a, wh1b, bh1, wh2, bh2,
                   h_o, x_o):
  hv = h[...]
  agg = p0[...] + p1[...]
  u = _silu(jnp.dot(hv, wh1a[...], preferred_element_type=jnp.float32)
            + jnp.dot(agg, wh1b[...], preferred_element_type=jnp.float32)
            + bh1[...])
  h_o[...] = hv + jnp.dot(u, wh2[...], preferred_element_type=jnp.float32) + bh2[...]
  q = q0[...] + q1[...]
  cnt = q[:, 3:4]
  xv = xa[...]
  mask = xv[:, 4:5]
  upd = q[:, 0:16] * (mask / jnp.maximum(cnt, 1.0))
  lane = lax.broadcasted_iota(jnp.int32, (BN, 16), 1)
  x_o[...] = xv + jnp.where(lane < 3, upd, 0.0)


def _node_upd(h, xa, p0, p1, q0, q1, w):
  grid = NPAD // BN
  nb = pl.BlockSpec((BN, HID), lambda i: (i, 0))
  n16 = pl.BlockSpec((BN, 16), lambda i: (i, 0))
  return pl.pallas_call(
      _node_upd_body,
      grid=(grid,),
      in_specs=[nb, n16, nb, nb, nb, nb,
                _rep((HID, HID)), _rep((HID, HID)), _rep((1, HID)),
                _rep((HID, HID)), _rep((1, HID))],
      out_specs=[nb, n16],
      out_shape=[jax.ShapeDtypeStruct((NPAD, HID), jnp.float32),
                 jax.ShapeDtypeStruct((NPAD, 16), jnp.float32)],
  )(h, xa, p0, p1, q0, q1, w["wh1a"], w["wh1b"], w["bh1"], w["wh2"], w["bh2"])


def _edge_enc_body(g1x, g2x, emb, m1w, m1b, m2w, m2b, e_o):
  lane = lax.broadcasted_iota(jnp.int32, (BE, 16), 1)
  diff = jnp.where(lane < 3,
                   g1x[:, HID:HID + 16] - g2x[:, HID:HID + 16], 0.0)
  el = jnp.sqrt(jnp.sum(diff * diff, axis=1, keepdims=True) + 1e-12)
  d = jax.nn.relu(el * m1w[...] + m1b[...])
  d = jnp.dot(d, m2w[...], preferred_element_type=jnp.float32) + m2b[...]
  e_o[...] = d * emb[...]


def _edge_enc(g1x, g2x, emb, m1w, m1b, m2w, m2b):
  grid = EPAD // BE
  eb = pl.BlockSpec((BE, HID), lambda i: (i, 0))
  e256 = pl.BlockSpec((BE, 256), lambda i: (i, 0))
  return pl.pallas_call(
      _edge_enc_body,
      grid=(grid,),
      in_specs=[e256, e256, eb, _rep((1, HID)), _rep((1, HID)),
                _rep((HID, HID)), _rep((1, HID))],
      out_specs=eb,
      out_shape=jax.ShapeDtypeStruct((EPAD, HID), jnp.float32),
  )(g1x, g2x, emb, m1w, m1b, m2w, m2b)


def _edge_len_body(g1x, g2x, e_o):
  lane = lax.broadcasted_iota(jnp.int32, (BE, 16), 1)
  diff = jnp.where(lane < 3,
                   g1x[:, HID:HID + 16] - g2x[:, HID:HID + 16], 0.0)
  el = jnp.sqrt(jnp.sum(diff * diff, axis=1, keepdims=True) + 1e-12)
  e_o[...] = jnp.where(lane == 0, el, 0.0)


def _edge_len(g1x, g2x):
  grid = EPAD // BE
  e256 = pl.BlockSpec((BE, 256), lambda i: (i, 0))
  e16 = pl.BlockSpec((BE, 16), lambda i: (i, 0))
  return pl.pallas_call(
      _edge_len_body, grid=(grid,), in_specs=[e256, e256], out_specs=e16,
      out_shape=jax.ShapeDtypeStruct((EPAD, 16), jnp.float32),
  )(g1x, g2x)


def _init_body(ni, nemb, wnl, bnl, tsf, freqs, wt1, bt1, wt2, bt2,
               temb_tab, wtl, btl, h_o):
  ai = ni[:, 0:1]
  oh_a = (lax.broadcasted_iota(jnp.int32, (BN, 128), 1) == ai).astype(jnp.float32)
  sil_tab = _silu(nemb[...])
  nv = jnp.dot(jnp.dot(oh_a, sil_tab, preferred_element_type=jnp.float32),
               wnl[...], preferred_element_type=jnp.float32) + bnl[...]
  bi = ni[:, 1:2]
  oh_b = (lax.broadcasted_iota(jnp.int32, (BN, 64), 1) == bi).astype(jnp.float32)
  t0 = jnp.dot(oh_b, tsf[...], preferred_element_type=jnp.float32)[:, 0:1]
  ang = t0 * freqs[...]
  te = jnp.concatenate([jnp.sin(ang), jnp.cos(ang)], axis=1)
  te = _silu(jnp.dot(te, wt1[...], preferred_element_type=jnp.float32) + bt1[...])
  te = jnp.dot(te, wt2[...], preferred_element_type=jnp.float32) + bt2[...]
  mrows = jnp.dot(_silu(temb_tab[...]), wtl[...],
                  preferred_element_type=jnp.float32) + btl[...]
  tif = (ni[:, 2:3] == 1).astype(jnp.float32)
  memb = mrows[0:1, :] + tif * (mrows[1:2, :] - mrows[0:1, :])
  h_o[...] = jnp.concatenate([nv, te, memb], axis=1)


def _init_h(ni, p):
  grid = NPAD // BN
  return pl.pallas_call(
      _init_body,
      grid=(grid,),
      in_specs=[pl.BlockSpec((BN, 8), lambda i: (i, 0)),
                _rep((128, 64)), _rep((64, 64)), _rep((1, 64)),
                _rep((64, 8)), _rep((1, 16)),
                _rep((32, 32)), _rep((1, 32)), _rep((32, 32)), _rep((1, 32)),
                _rep((8, 32)), _rep((32, 32)), _rep((1, 32))],
      out_specs=pl.BlockSpec((BN, HID), lambda i: (i, 0)),
      out_shape=jax.ShapeDtypeStruct((NPAD, HID), jnp.float32),
  )(ni, p["nemb"], p["wnl"], p["bnl"], p["tsf"], p["freqs"],
    p["wt1"], p["bt1"], p["wt2"], p["bt2"], p["temb_tab"], p["wtl"], p["btl"])


def _prep_egcl(p):
  we1 = p["e1"]["w"]
  enf = we1.shape[0] - 2 * HID - 1
  wea = we1[2 * HID + 1:]
  if enf == 1:
    wea = jnp.pad(wea, ((0, 15), (0, 0)))
  return {
      "wa": we1[0:HID], "wb": we1[HID:2 * HID],
      "wd": we1[2 * HID:2 * HID + 1], "wea": wea,
      "b1": p["e1"]["b"][None, :],
      "we2": p["e2"]["w"], "b2": p["e2"]["b"][None, :],
      "watt": jnp.pad(p["att"]["w"], ((0, 0), (0, 7))),
      "batt": jnp.pad(p["att"]["b"][None, :], ((0, 0), (0, 7))),
      "wx1": p["x1"]["w"], "bx1": p["x1"]["b"][None, :],
      "wx2": jnp.pad(p["x2"]["w"], ((0, 0), (0, 7))),
      "bx2": jnp.pad(p["x2"]["b"][None, :], ((0, 0), (0, 7))),
      "wh1a": p["h1"]["w"][0:HID], "wh1b": p["h1"]["w"][HID:],
      "bh1": p["h1"]["b"][None, :],
      "wh2": p["h2"]["w"], "bh2": p["h2"]["b"][None, :],
  }


def _pad_edges(idx, fill):
  return jnp.concatenate(
      [idx.astype(jnp.int32),
       jnp.full((EPAD - N_EDGES,), fill, jnp.int32)])


def kernel(atom_type, pos, bond_index, bond_type, batch, graph_idx,
           time_step, template_mask, edge_index_a, params):
  del graph_idx
  row_b = _pad_edges(bond_index[0], DUMMY)
  col_b = _pad_edges(bond_index[1], 0)
  row_a = _pad_edges(edge_index_a[0], DUMMY)
  col_a = _pad_edges(edge_index_a[1], 0)
  typ = _pad_edges(bond_type, 0)

  maskf = template_mask.astype(jnp.float32)
  xa = jnp.zeros((NPAD, 16), jnp.float32)
  xa = xa.at[:N_NODES, 0:3].set(pos)
  xa = xa.at[:N_NODES, 4].set(maskf)

  ni = jnp.zeros((NPAD, 8), jnp.int32)
  ni = ni.at[:N_NODES, 0].set(atom_type.astype(jnp.int32))
  ni = ni.at[:N_NODES, 1].set(batch.astype(jnp.int32))
  ni = ni.at[:N_NODES, 2].set(template_mask.astype(jnp.int32))

  half = 16
  scale = math.log(10000.0) / (half - 1)
  freqs = jnp.exp(jnp.arange(half, dtype=jnp.float32) * -scale)[None, :]
  tsf = jnp.zeros((64, 8), jnp.float32).at[:, 0].set(
      time_step.astype(jnp.float32))

  ip = {
      "nemb": jnp.pad(params["node_emb"], ((0, 28), (0, 0))),
      "wnl": params["node_lin"]["w"], "bnl": params["node_lin"]["b"][None, :],
      "tsf": tsf, "freqs": freqs,
      "wt1": params["t1"]["w"], "bt1": params["t1"]["b"][None, :],
      "wt2": params["t2"]["w"], "bt2": params["t2"]["b"][None, :],
      "temb_tab": jnp.pad(params["tmpl_emb"], ((0, 6), (0, 0))),
      "wtl": params["tmpl_lin"]["w"], "btl": params["tmpl_lin"]["b"][None, :],
  }

  enc1 = [_prep_egcl(p) for p in params["enc1"]]
  encc = [_prep_egcl(p) for p in params["enc_cross"]]
  enc2 = [_prep_egcl(p) for p in params["enc2"]]

  ee1 = params["edge_enc"]
  ee2 = params["edge_enc2"]

  h = _init_h(ni, ip)

  emb1_pad = jnp.pad(ee1["emb"], ((0, 28), (0, 0)))
  emb2_pad = jnp.pad(ee2["emb"], ((0, 28), (0, 0)))
  emb_rows1, emb_rows2 = _sc_gather([emb1_pad, emb2_pad], [typ, typ])

  def gather_layer(h, xa, w, row, col):
    ta, tb = _node_pre(h, xa, w["wa"], w["wb"])
    return _sc_gather([ta, tb], [row, col])

  def finish_layer(h, xa, w, row, g1x, g2x, eat):
    mm, pay = _edge_mlp(g1x, g2x, eat, w)
    agg_h, agg_x = _sc_scatter(mm, pay, row)
    return _node_upd(h, xa, agg_h[:NPAD], agg_h[NPAD:],
                     agg_x[:NPAD], agg_x[NPAD:], w)

  e1buf = None
  for li, w in enumerate(enc1):
    g1x, g2x = gather_layer(h, xa, w, row_b, col_b)
    if li == 0:
      e1buf = _edge_enc(g1x, g2x, emb_rows1, ee1["m1"]["w"],
                        ee1["m1"]["b"][None, :], ee1["m2"]["w"],
                        ee1["m2"]["b"][None, :])
    h, xa = finish_layer(h, xa, w, row_b, g1x, g2x, e1buf)

  ela = None
  for li, w in enumerate(encc):
    g1x, g2x = gather_layer(h, xa, w, row_a, col_a)
    if li == 0:
      ela = _edge_len(g1x, g2x)
    h, xa = finish_layer(h, xa, w, row_a, g1x, g2x, ela)

  e2buf = None
  for li, w in enumerate(enc2):
    g1x, g2x = gather_layer(h, xa, w, row_b, col_b)
    if li == 0:
      e2buf = _edge_enc(g1x, g2x, emb_rows2, ee2["m1"]["w"],
                        ee2["m1"]["b"][None, :], ee2["m2"]["w"],
                        ee2["m2"]["b"][None, :])
    h, xa = finish_layer(h, xa, w, row_b, g1x, g2x, e2buf)

  return xa[:N_NODES, 0:3] - pos


# speedup vs baseline: 1.9301x; 1.0869x over previous
import functools
import math

import jax
import jax.numpy as jnp
from jax import lax
from jax.experimental import pallas as pl
from jax.experimental.pallas import tpu as pltpu
from jax.experimental.pallas import tpu_sc as plsc

N_NODES = 10000
N_EDGES = 160000
HID = 128

NC = 2
NS = 16
NW = NC * NS
CHUNK = 128

NPAD = 10240
DUMMY = N_NODES
EPAD = 163840
E_PER_TILE = EPAD // NW
CH_PER_TILE = E_PER_TILE // CHUNK
ROWS_PER_TILE = NPAD // NS
EXPORT_CH = ROWS_PER_TILE // CHUNK

BE = 1024
BN = 1024

_mesh = functools.partial(
    plsc.VectorSubcoreMesh, core_axis_name="c", subcore_axis_name="s",
    num_cores=NC, num_subcores=NS)


CHUNK_G = 80
CHG_PER_TILE = E_PER_TILE // CHUNK_G
GSLOTS = 2


def _gather_body(n_tasks, widths, *refs):
  tabs = refs[0:n_tasks]
  idxs = refs[n_tasks:2 * n_tasks]
  outs = refs[2 * n_tasks:3 * n_tasks]
  scr = refs[3 * n_tasks:]
  idx_all = scr[0:n_tasks]
  rb = scr[n_tasks:n_tasks + n_tasks * GSLOTS]
  sg = scr[n_tasks + n_tasks * GSLOTS:n_tasks + 2 * n_tasks * GSLOTS]
  sw = scr[n_tasks + 2 * n_tasks * GSLOTS:n_tasks + 3 * n_tasks * GSLOTS]
  si = scr[n_tasks + 3 * n_tasks * GSLOTS]
  cid = lax.axis_index("c")
  sid = lax.axis_index("s")
  wid = sid * NC + cid
  base = wid * E_PER_TILE
  base_c = wid * CHG_PER_TILE

  d = [pltpu.async_copy(idxs[k].at[pl.ds(base_c, CHG_PER_TILE)], idx_all[k], si)
       for k in range(n_tasks)]
  for x in d:
    x.wait()

  def it(c2, carry):
    gd = []
    for s in range(GSLOTS):
      c = GSLOTS * c2 + s
      for k in range(n_tasks):
        gd.append(pltpu.async_copy(
            tabs[k].at[idx_all[k].at[c]], rb[k * GSLOTS + s],
            sg[k * GSLOTS + s]))
    wd = []
    for s in range(GSLOTS):
      c = GSLOTS * c2 + s
      eb = base + c * CHUNK_G
      for k in range(n_tasks):
        gd[s * n_tasks + k].wait()
        wd.append(pltpu.async_copy(
            rb[k * GSLOTS + s], outs[k].at[pl.ds(eb, CHUNK_G)],
            sw[k * GSLOTS + s]))
    for x in wd:
      x.wait()
    return carry

  lax.fori_loop(0, CHG_PER_TILE // GSLOTS, it, 0)


def _sc_gather(tables, indices):
  n = len(tables)
  widths = [t.shape[1] for t in tables]
  out_type = [jax.ShapeDtypeStruct((EPAD, w), jnp.float32) for w in widths]
  scratch = ([pltpu.VMEM((CHG_PER_TILE, CHUNK_G), jnp.int32) for _ in range(n)]
             + [pltpu.VMEM((CHUNK_G, w), jnp.float32)
                for w in widths for _ in range(GSLOTS)]
             + [pltpu.SemaphoreType.DMA for _ in range(2 * n * GSLOTS + 1)])
  fn = pl.kernel(
      functools.partial(_gather_body, n, tuple(widths)),
      out_type=out_type, mesh=_mesh(), scratch_types=scratch)
  idx2 = [i.reshape(-1, CHUNK_G) for i in indices]
  return fn(*tables, *idx2)


CS = 64
CHS_PER_TILE = E_PER_TILE // CS
EXPORT_CHS = ROWS_PER_TILE // CS


def _scatter_body(mm, pay, row2, out_h, out_x, acc, idx_all,
                  bm0, bm1, bp0, bp1, sl0, sl1, sa0, sa1, si):
  cid = lax.axis_index("c")
  sid = lax.axis_index("s")
  wid = sid * NC + cid
  base = wid * E_PER_TILE
  base_c = wid * CHS_PER_TILE
  bms = (bm0, bm1)
  bps = (bp0, bp1)
  sls = (sl0, sl1)
  sas = (sa0, sa1)

  pltpu.async_copy(row2.at[pl.ds(base_c, CHS_PER_TILE)], idx_all, si).wait()

  zero = jnp.zeros((16,), jnp.float32)

  def zinit(buf):
    def body(i, carry):
      for j in range(HID // 16):
        buf[i, pl.ds(j * 16, 16)] = zero
      return carry
    return body

  def zfill(k, carry):
    r = sid * ROWS_PER_TILE + k * CS
    pltpu.sync_copy(bm0, acc.at[pl.ds(r, CS)])
    return carry

  def export(out):
    def body(k, carry):
      r = sid * ROWS_PER_TILE + k * CS
      pltpu.sync_copy(acc.at[pl.ds(r, CS)], bm0)
      pltpu.sync_copy(bm0, out.at[pl.ds(cid * NPAD + r, CS)])
      return carry
    return body

  lax.fori_loop(0, CS, zinit(bm0), 0)
  lax.fori_loop(0, EXPORT_CHS, zfill, 0)
  plsc.subcore_barrier()

  def it_mm(c2, carry):
    dl = []
    for s in range(2):
      c = 2 * c2 + s
      dl.append(pltpu.async_copy(mm.at[pl.ds(base + c * CS, CS)],
                                 bms[s], sls[s]))
    da = []
    for s in range(2):
      c = 2 * c2 + s
      dl[s].wait()
      da.append(pltpu.async_copy(bms[s], acc.at[idx_all.at[c]], sas[s],
                                 add=True))
    for x in da:
      x.wait()
    return carry
  lax.fori_loop(0, CHS_PER_TILE // 2, it_mm, 0)
  plsc.subcore_barrier()
  lax.fori_loop(0, EXPORT_CHS, export(out_h), 0)
  plsc.subcore_barrier()

  lax.fori_loop(0, CS, zinit(bm0), 0)
  lax.fori_loop(0, EXPORT_CHS, zfill, 0)
  plsc.subcore_barrier()
  lax.fori_loop(0, CS, zinit(bm0), 0)
  lax.fori_loop(0, CS, zinit(bm1), 0)

  def it_pay(c2, carry):
    dl = []
    for s in range(2):
      c = 2 * c2 + s
      dl.append(pltpu.async_copy(pay.at[pl.ds(base + c * CS, CS)],
                                 bps[s], sls[s]))
    da = []
    for s in range(2):
      c = 2 * c2 + s
      dl[s].wait()

      def fill(i, carry2, _bm=bms[s], _bp=bps[s]):
        _bm[i, pl.ds(0, 16)] = _bp[i]
        return carry2
      lax.fori_loop(0, CS, fill, 0)
      da.append(pltpu.async_copy(bms[s], acc.at[idx_all.at[c]], sas[s],
                                 add=True))
    for x in da:
      x.wait()
    return carry
  lax.fori_loop(0, CHS_PER_TILE // 2, it_pay, 0)
  plsc.subcore_barrier()
  lax.fori_loop(0, EXPORT_CHS, export(out_x), 0)


def _sc_scatter(mm, pay, row):
  out_type = [jax.ShapeDtypeStruct((NC * NPAD, HID), jnp.float32),
              jax.ShapeDtypeStruct((NC * NPAD, HID), jnp.float32)]
  scratch = [
      pltpu.VMEM_SHARED((NPAD, HID), jnp.float32),
      pltpu.VMEM((CHS_PER_TILE, CS), jnp.int32),
      pltpu.VMEM((CS, HID), jnp.float32),
      pltpu.VMEM((CS, HID), jnp.float32),
      pltpu.VMEM((CS, 16), jnp.float32),
      pltpu.VMEM((CS, 16), jnp.float32),
      pltpu.SemaphoreType.DMA,
      pltpu.SemaphoreType.DMA,
      pltpu.SemaphoreType.DMA,
      pltpu.SemaphoreType.DMA,
      pltpu.SemaphoreType.DMA,
  ]
  fn = pl.kernel(_scatter_body, out_type=out_type, mesh=_mesh(),
                 scratch_types=scratch)
  return fn(mm, pay, row.reshape(-1, CS))


def _silu(v):
  return v * jax.nn.sigmoid(v)


def _rep(shape):
  nd = len(shape)
  return pl.BlockSpec(shape, lambda i: (0,) * nd)


def _node_pre_body(h, xa, wa, wb, ta, tb):
  hv = h[...]
  xav = xa[...]
  z = jnp.zeros((BN, 112), jnp.float32)
  ta[...] = jnp.concatenate(
      [jnp.dot(hv, wa[...], preferred_element_type=jnp.float32), xav, z], axis=1)
  tb[...] = jnp.concatenate(
      [jnp.dot(hv, wb[...], preferred_element_type=jnp.float32), xav, z], axis=1)


def _node_pre(h, xa, wa, wb):
  grid = NPAD // BN
  return pl.pallas_call(
      _node_pre_body,
      grid=(grid,),
      in_specs=[pl.BlockSpec((BN, HID), lambda i: (i, 0)),
                pl.BlockSpec((BN, 16), lambda i: (i, 0)),
                _rep((HID, HID)), _rep((HID, HID))],
      out_specs=[pl.BlockSpec((BN, 256), lambda i: (i, 0))] * 2,
      out_shape=[jax.ShapeDtypeStruct((NPAD, 256), jnp.float32)] * 2,
  )(h, xa, wa, wb)


def _edge_mlp_body(g1x, g2x, eat, wea, b1, wd, we2, b2,
                   watt, batt, wx1, bx1, wx2, bx2, mm_o, pay_o):
  lane = lax.broadcasted_iota(jnp.int32, (BE, 16), 1)
  g1v = g1x[...]
  g2v = g2x[...]
  diff = jnp.where(lane < 3, g1v[:, HID:HID + 16] - g2v[:, HID:HID + 16], 0.0)
  d2 = jnp.sum(diff * diff, axis=1, keepdims=True)
  pre = (g1v[:, 0:HID] + g2v[:, 0:HID] + d2 * wd[...]
         + jnp.dot(eat[...], wea[...], preferred_element_type=jnp.float32)
         + b1[...])
  m = _silu(pre)
  m = _silu(jnp.dot(m, we2[...], preferred_element_type=jnp.float32) + b2[...])
  att = jax.nn.sigmoid(
      jnp.dot(m, watt[...], preferred_element_type=jnp.float32) + batt[...])[:, 0:1]
  mm = m * att
  t = _silu(jnp.dot(mm, wx1[...], preferred_element_type=jnp.float32) + bx1[...])
  tx = (jnp.dot(t, wx2[...], preferred_element_type=jnp.float32) + bx2[...])[:, 0:1]
  mm_o[...] = mm
  pay_o[...] = diff * tx + (lane == 3).astype(jnp.float32)


def _edge_mlp(g1x, g2x, eat, w):
  ew = eat.shape[1]
  grid = EPAD // BE
  eb = pl.BlockSpec((BE, HID), lambda i: (i, 0))
  e256 = pl.BlockSpec((BE, 256), lambda i: (i, 0))
  e16 = pl.BlockSpec((BE, 16), lambda i: (i, 0))
  return pl.pallas_call(
      _edge_mlp_body,
      grid=(grid,),
      in_specs=[e256, e256, pl.BlockSpec((BE, ew), lambda i: (i, 0)),
                _rep((ew, HID)), _rep((1, HID)), _rep((1, HID)),
                _rep((HID, HID)), _rep((1, HID)),
                _rep((HID, 8)), _rep((1, 8)),
                _rep((HID, HID)), _rep((1, HID)),
                _rep((HID, 8)), _rep((1, 8))],
      out_specs=[eb, e16],
      out_shape=[jax.ShapeDtypeStruct((EPAD, HID), jnp.float32),
                 jax.ShapeDtypeStruct((EPAD, 16), jnp.float32)],
  )(g1x, g2x, eat, w["wea"], w["b1"], w["wd"], w["we2"], w["b2"],
    w["watt"], w["batt"], w["wx1"], w["bx1"], w["wx2"], w["bx2"])


def _node_upd_body(h, xa, p0, p1, q0, q1, wh1a, wh1b, bh1, wh2, bh2,
                   h_o, x_o):
  hv = h[...]
  agg = p0[...] + p1[...]
  u = _silu(jnp.dot(hv, wh1a[...], preferred_element_type=jnp.float32)
            + jnp.dot(agg, wh1b[...], preferred_element_type=jnp.float32)
            + bh1[...])
  h_o[...] = hv + jnp.dot(u, wh2[...], preferred_element_type=jnp.float32) + bh2[...]
  q = q0[...] + q1[...]
  cnt = q[:, 3:4]
  xv = xa[...]
  mask = xv[:, 4:5]
  upd = q[:, 0:16] * (mask / jnp.maximum(cnt, 1.0))
  lane = lax.broadcasted_iota(jnp.int32, (BN, 16), 1)
  x_o[...] = xv + jnp.where(lane < 3, upd, 0.0)


def _node_upd(h, xa, p0, p1, q0, q1, w):
  grid = NPAD // BN
  nb = pl.BlockSpec((BN, HID), lambda i: (i, 0))
  n16 = pl.BlockSpec((BN, 16), lambda i: (i, 0))
  return pl.pallas_call(
      _node_upd_body,
      grid=(grid,),
      in_specs=[nb, n16, nb, nb, nb, nb,
                _rep((HID, HID)), _rep((HID, HID)), _rep((1, HID)),
                _rep((HID, HID)), _rep((1, HID))],
      out_specs=[nb, n16],
      out_shape=[jax.ShapeDtypeStruct((NPAD, HID), jnp.float32),
                 jax.ShapeDtypeStruct((NPAD, 16), jnp.float32)],
  )(h, xa, p0, p1, q0, q1, w["wh1a"], w["wh1b"], w["bh1"], w["wh2"], w["bh2"])


def _edge_enc_body(g1x, g2x, emb, m1w, m1b, m2w, m2b, e_o):
  lane = lax.broadcasted_iota(jnp.int32, (BE, 16), 1)
  diff = jnp.where(lane < 3,
                   g1x[:, HID:HID + 16] - g2x[:, HID:HID + 16], 0.0)
  el = jnp.sqrt(jnp.sum(diff * diff, axis=1, keepdims=True) + 1e-12)
  d = jax.nn.relu(el * m1w[...] + m1b[...])
  d = jnp.dot(d, m2w[...], preferred_element_type=jnp.float32) + m2b[...]
  e_o[...] = d * emb[...]


def _edge_enc(g1x, g2x, emb, m1w, m1b, m2w, m2b):
  grid = EPAD // BE
  eb = pl.BlockSpec((BE, HID), lambda i: (i, 0))
  e256 = pl.BlockSpec((BE, 256), lambda i: (i, 0))
  return pl.pallas_call(
      _edge_enc_body,
      grid=(grid,),
      in_specs=[e256, e256, eb, _rep((1, HID)), _rep((1, HID)),
                _rep((HID, HID)), _rep((1, HID))],
      out_specs=eb,
      out_shape=jax.ShapeDtypeStruct((EPAD, HID), jnp.float32),
  )(g1x, g2x, emb, m1w, m1b, m2w, m2b)


def _edge_len_body(g1x, g2x, e_o):
  lane = lax.broadcasted_iota(jnp.int32, (BE, 16), 1)
  diff = jnp.where(lane < 3,
                   g1x[:, HID:HID + 16] - g2x[:, HID:HID + 16], 0.0)
  el = jnp.sqrt(jnp.sum(diff * diff, axis=1, keepdims=True) + 1e-12)
  e_o[...] = jnp.where(lane == 0, el, 0.0)


def _edge_len(g1x, g2x):
  grid = EPAD // BE
  e256 = pl.BlockSpec((BE, 256), lambda i: (i, 0))
  e16 = pl.BlockSpec((BE, 16), lambda i: (i, 0))
  return pl.pallas_call(
      _edge_len_body, grid=(grid,), in_specs=[e256, e256], out_specs=e16,
      out_shape=jax.ShapeDtypeStruct((EPAD, 16), jnp.float32),
  )(g1x, g2x)


def _init_body(ni, nemb, wnl, bnl, tsf, freqs, wt1, bt1, wt2, bt2,
               temb_tab, wtl, btl, h_o):
  ai = ni[:, 0:1]
  oh_a = (lax.broadcasted_iota(jnp.int32, (BN, 128), 1) == ai).astype(jnp.float32)
  sil_tab = _silu(nemb[...])
  nv = jnp.dot(jnp.dot(oh_a, sil_tab, preferred_element_type=jnp.float32),
               wnl[...], preferred_element_type=jnp.float32) + bnl[...]
  bi = ni[:, 1:2]
  oh_b = (lax.broadcasted_iota(jnp.int32, (BN, 64), 1) == bi).astype(jnp.float32)
  t0 = jnp.dot(oh_b, tsf[...], preferred_element_type=jnp.float32)[:, 0:1]
  ang = t0 * freqs[...]
  te = jnp.concatenate([jnp.sin(ang), jnp.cos(ang)], axis=1)
  te = _silu(jnp.dot(te, wt1[...], preferred_element_type=jnp.float32) + bt1[...])
  te = jnp.dot(te, wt2[...], preferred_element_type=jnp.float32) + bt2[...]
  mrows = jnp.dot(_silu(temb_tab[...]), wtl[...],
                  preferred_element_type=jnp.float32) + btl[...]
  tif = (ni[:, 2:3] == 1).astype(jnp.float32)
  memb = mrows[0:1, :] + tif * (mrows[1:2, :] - mrows[0:1, :])
  h_o[...] = jnp.concatenate([nv, te, memb], axis=1)


def _init_h(ni, p):
  grid = NPAD // BN
  return pl.pallas_call(
      _init_body,
      grid=(grid,),
      in_specs=[pl.BlockSpec((BN, 8), lambda i: (i, 0)),
                _rep((128, 64)), _rep((64, 64)), _rep((1, 64)),
                _rep((64, 8)), _rep((1, 16)),
                _rep((32, 32)), _rep((1, 32)), _rep((32, 32)), _rep((1, 32)),
                _rep((8, 32)), _rep((32, 32)), _rep((1, 32))],
      out_specs=pl.BlockSpec((BN, HID), lambda i: (i, 0)),
      out_shape=jax.ShapeDtypeStruct((NPAD, HID), jnp.float32),
  )(ni, p["nemb"], p["wnl"], p["bnl"], p["tsf"], p["freqs"],
    p["wt1"], p["bt1"], p["wt2"], p["bt2"], p["temb_tab"], p["wtl"], p["btl"])


def _prep_egcl(p):
  we1 = p["e1"]["w"]
  enf = we1.shape[0] - 2 * HID - 1
  wea = we1[2 * HID + 1:]
  if enf == 1:
    wea = jnp.pad(wea, ((0, 15), (0, 0)))
  return {
      "wa": we1[0:HID], "wb": we1[HID:2 * HID],
      "wd": we1[2 * HID:2 * HID + 1], "wea": wea,
      "b1": p["e1"]["b"][None, :],
      "we2": p["e2"]["w"], "b2": p["e2"]["b"][None, :],
      "watt": jnp.pad(p["att"]["w"], ((0, 0), (0, 7))),
      "batt": jnp.pad(p["att"]["b"][None, :], ((0, 0), (0, 7))),
      "wx1": p["x1"]["w"], "bx1": p["x1"]["b"][None, :],
      "wx2": jnp.pad(p["x2"]["w"], ((0, 0), (0, 7))),
      "bx2": jnp.pad(p["x2"]["b"][None, :], ((0, 0), (0, 7))),
      "wh1a": p["h1"]["w"][0:HID], "wh1b": p["h1"]["w"][HID:],
      "bh1": p["h1"]["b"][None, :],
      "wh2": p["h2"]["w"], "bh2": p["h2"]["b"][None, :],
  }


def _pad_edges(idx, fill):
  return jnp.concatenate(
      [idx.astype(jnp.int32),
       jnp.full((EPAD - N_EDGES,), fill, jnp.int32)])


def kernel(atom_type, pos, bond_index, bond_type, batch, graph_idx,
           time_step, template_mask, edge_index_a, params):
  del graph_idx
  row_b = _pad_edges(bond_index[0], DUMMY)
  col_b = _pad_edges(bond_index[1], 0)
  row_a = _pad_edges(edge_index_a[0], DUMMY)
  col_a = _pad_edges(edge_index_a[1], 0)
  typ = _pad_edges(bond_type, 0)

  maskf = template_mask.astype(jnp.float32)
  xa = jnp.zeros((NPAD, 16), jnp.float32)
  xa = xa.at[:N_NODES, 0:3].set(pos)
  xa = xa.at[:N_NODES, 4].set(maskf)

  ni = jnp.zeros((NPAD, 8), jnp.int32)
  ni = ni.at[:N_NODES, 0].set(atom_type.astype(jnp.int32))
  ni = ni.at[:N_NODES, 1].set(batch.astype(jnp.int32))
  ni = ni.at[:N_NODES, 2].set(template_mask.astype(jnp.int32))

  half = 16
  scale = math.log(10000.0) / (half - 1)
  freqs = jnp.exp(jnp.arange(half, dtype=jnp.float32) * -scale)[None, :]
  tsf = jnp.zeros((64, 8), jnp.float32).at[:, 0].set(
      time_step.astype(jnp.float32))

  ip = {
      "nemb": jnp.pad(params["node_emb"], ((0, 28), (0, 0))),
      "wnl": params["node_lin"]["w"], "bnl": params["node_lin"]["b"][None, :],
      "tsf": tsf, "freqs": freqs,
      "wt1": params["t1"]["w"], "bt1": params["t1"]["b"][None, :],
      "wt2": params["t2"]["w"], "bt2": params["t2"]["b"][None, :],
      "temb_tab": jnp.pad(params["tmpl_emb"], ((0, 6), (0, 0))),
      "wtl": params["tmpl_lin"]["w"], "btl": params["tmpl_lin"]["b"][None, :],
  }

  enc1 = [_prep_egcl(p) for p in params["enc1"]]
  encc = [_prep_egcl(p) for p in params["enc_cross"]]
  enc2 = [_prep_egcl(p) for p in params["enc2"]]

  ee1 = params["edge_enc"]
  ee2 = params["edge_enc2"]

  h = _init_h(ni, ip)

  emb1_pad = jnp.pad(ee1["emb"], ((0, 28), (0, 0)))
  emb2_pad = jnp.pad(ee2["emb"], ((0, 28), (0, 0)))
  emb_rows1, emb_rows2 = _sc_gather([emb1_pad, emb2_pad], [typ, typ])

  def gather_layer(h, xa, w, row, col):
    ta, tb = _node_pre(h, xa, w["wa"], w["wb"])
    return _sc_gather([ta, tb], [row, col])

  def finish_layer(h, xa, w, row, g1x, g2x, eat):
    mm, pay = _edge_mlp(g1x, g2x, eat, w)
    agg_h, agg_x = _sc_scatter(mm, pay, row)
    return _node_upd(h, xa, agg_h[:NPAD], agg_h[NPAD:],
                     agg_x[:NPAD], agg_x[NPAD:], w)

  e1buf = None
  for li, w in enumerate(enc1):
    g1x, g2x = gather_layer(h, xa, w, row_b, col_b)
    if li == 0:
      e1buf = _edge_enc(g1x, g2x, emb_rows1, ee1["m1"]["w"],
                        ee1["m1"]["b"][None, :], ee1["m2"]["w"],
                        ee1["m2"]["b"][None, :])
    h, xa = finish_layer(h, xa, w, row_b, g1x, g2x, e1buf)

  ela = None
  for li, w in enumerate(encc):
    g1x, g2x = gather_layer(h, xa, w, row_a, col_a)
    if li == 0:
      ela = _edge_len(g1x, g2x)
    h, xa = finish_layer(h, xa, w, row_a, g1x, g2x, ela)

  e2buf = None
  for li, w in enumerate(enc2):
    g1x, g2x = gather_layer(h, xa, w, row_b, col_b)
    if li == 0:
      e2buf = _edge_enc(g1x, g2x, emb_rows2, ee2["m1"]["w"],
                        ee2["m1"]["b"][None, :], ee2["m2"]["w"],
                        ee2["m2"]["b"][None, :])
    h, xa = finish_layer(h, xa, w, row_b, g1x, g2x, e2buf)

  return xa[:N_NODES, 0:3] - pos


# speedup vs baseline: 2.0149x; 1.0439x over previous
import functools
import math

import jax
import jax.numpy as jnp
from jax import lax
from jax.experimental import pallas as pl
from jax.experimental.pallas import tpu as pltpu
from jax.experimental.pallas import tpu_sc as plsc

N_NODES = 10000
N_EDGES = 160000
HID = 128

NC = 2
NS = 16
NW = NC * NS
CHUNK = 128

NPAD = 10240
DUMMY = N_NODES
EPAD = 163840
E_PER_TILE = EPAD // NW
CH_PER_TILE = E_PER_TILE // CHUNK
ROWS_PER_TILE = NPAD // NS
EXPORT_CH = ROWS_PER_TILE // CHUNK

BE = 1024
BN = 1024

_mesh = functools.partial(
    plsc.VectorSubcoreMesh, core_axis_name="c", subcore_axis_name="s",
    num_cores=NC, num_subcores=NS)


CHUNK_G = 80
CHG_PER_TILE = E_PER_TILE // CHUNK_G
GSLOTS = 2


def _gather_body(n_tasks, widths, *refs):
  tabs = refs[0:n_tasks]
  idxs = refs[n_tasks:2 * n_tasks]
  outs = refs[2 * n_tasks:3 * n_tasks]
  scr = refs[3 * n_tasks:]
  idx_all = scr[0:n_tasks]
  rb = scr[n_tasks:n_tasks + n_tasks * GSLOTS]
  sg = scr[n_tasks + n_tasks * GSLOTS:n_tasks + 2 * n_tasks * GSLOTS]
  sw = scr[n_tasks + 2 * n_tasks * GSLOTS:n_tasks + 3 * n_tasks * GSLOTS]
  si = scr[n_tasks + 3 * n_tasks * GSLOTS]
  cid = lax.axis_index("c")
  sid = lax.axis_index("s")
  wid = sid * NC + cid
  base = wid * E_PER_TILE
  base_c = wid * CHG_PER_TILE

  d = [pltpu.async_copy(idxs[k].at[pl.ds(base_c, CHG_PER_TILE)], idx_all[k], si)
       for k in range(n_tasks)]
  for x in d:
    x.wait()

  def drain_wb(s):
    for k in range(n_tasks):
      pltpu.make_async_copy(
          rb[k * GSLOTS + s], outs[k].at[pl.ds(base, CHUNK_G)],
          sw[k * GSLOTS + s]).wait()

  def it(c2, carry):
    gd = []
    for s in range(GSLOTS):
      c = GSLOTS * c2 + s

      @pl.when(c2 > 0)
      def _(s=s):
        drain_wb(s)

      for k in range(n_tasks):
        gd.append(pltpu.async_copy(
            tabs[k].at[idx_all[k].at[c]], rb[k * GSLOTS + s],
            sg[k * GSLOTS + s]))
    for s in range(GSLOTS):
      c = GSLOTS * c2 + s
      eb = base + c * CHUNK_G
      for k in range(n_tasks):
        gd[s * n_tasks + k].wait()
        pltpu.async_copy(
            rb[k * GSLOTS + s], outs[k].at[pl.ds(eb, CHUNK_G)],
            sw[k * GSLOTS + s])
    return carry

  lax.fori_loop(0, CHG_PER_TILE // GSLOTS, it, 0)
  for s in range(GSLOTS):
    drain_wb(s)


def _sc_gather(tables, indices):
  n = len(tables)
  widths = [t.shape[1] for t in tables]
  out_type = [jax.ShapeDtypeStruct((EPAD, w), jnp.float32) for w in widths]
  scratch = ([pltpu.VMEM((CHG_PER_TILE, CHUNK_G), jnp.int32) for _ in range(n)]
             + [pltpu.VMEM((CHUNK_G, w), jnp.float32)
                for w in widths for _ in range(GSLOTS)]
             + [pltpu.SemaphoreType.DMA for _ in range(2 * n * GSLOTS + 1)])
  fn = pl.kernel(
      functools.partial(_gather_body, n, tuple(widths)),
      out_type=out_type, mesh=_mesh(), scratch_types=scratch)
  idx2 = [i.reshape(-1, CHUNK_G) for i in indices]
  return fn(*tables, *idx2)


CS = 64
CHS_PER_TILE = E_PER_TILE // CS
EXPORT_CHS = ROWS_PER_TILE // CS


def _scatter_body(mm, pay, row2, out_h, out_x, acc, idx_all,
                  bm0, bm1, bp0, bp1, sl0, sl1, sa0, sa1, si):
  cid = lax.axis_index("c")
  sid = lax.axis_index("s")
  wid = sid * NC + cid
  base = wid * E_PER_TILE
  base_c = wid * CHS_PER_TILE
  bms = (bm0, bm1)
  bps = (bp0, bp1)
  sls = (sl0, sl1)
  sas = (sa0, sa1)

  pltpu.async_copy(row2.at[pl.ds(base_c, CHS_PER_TILE)], idx_all, si).wait()

  zero = jnp.zeros((16,), jnp.float32)

  def zinit(buf):
    def body(i, carry):
      for j in range(HID // 16):
        buf[i, pl.ds(j * 16, 16)] = zero
      return carry
    return body

  def zfill(k, carry):
    r = sid * ROWS_PER_TILE + k * CS
    pltpu.sync_copy(bm0, acc.at[pl.ds(r, CS)])
    return carry

  def export(out):
    def body(k, carry):
      r = sid * ROWS_PER_TILE + k * CS
      pltpu.sync_copy(acc.at[pl.ds(r, CS)], bm0)
      pltpu.sync_copy(bm0, out.at[pl.ds(cid * NPAD + r, CS)])
      return carry
    return body

  lax.fori_loop(0, CS, zinit(bm0), 0)
  lax.fori_loop(0, EXPORT_CHS, zfill, 0)
  plsc.subcore_barrier()

  def it_mm(c2, carry):
    dl = []
    for s in range(2):
      c = 2 * c2 + s
      dl.append(pltpu.async_copy(mm.at[pl.ds(base + c * CS, CS)],
                                 bms[s], sls[s]))
    da = []
    for s in range(2):
      c = 2 * c2 + s
      dl[s].wait()
      da.append(pltpu.async_copy(bms[s], acc.at[idx_all.at[c]], sas[s],
                                 add=True))
    for x in da:
      x.wait()
    return carry
  lax.fori_loop(0, CHS_PER_TILE // 2, it_mm, 0)
  plsc.subcore_barrier()
  lax.fori_loop(0, EXPORT_CHS, export(out_h), 0)
  plsc.subcore_barrier()

  lax.fori_loop(0, CS, zinit(bm0), 0)
  lax.fori_loop(0, EXPORT_CHS, zfill, 0)
  plsc.subcore_barrier()
  lax.fori_loop(0, CS, zinit(bm0), 0)
  lax.fori_loop(0, CS, zinit(bm1), 0)

  def it_pay(c2, carry):
    dl = []
    for s in range(2):
      c = 2 * c2 + s
      dl.append(pltpu.async_copy(pay.at[pl.ds(base + c * CS, CS)],
                                 bps[s], sls[s]))
    da = []
    for s in range(2):
      c = 2 * c2 + s
      dl[s].wait()

      def fill(i, carry2, _bm=bms[s], _bp=bps[s]):
        _bm[i, pl.ds(0, 16)] = _bp[i]
        return carry2
      lax.fori_loop(0, CS, fill, 0)
      da.append(pltpu.async_copy(bms[s], acc.at[idx_all.at[c]], sas[s],
                                 add=True))
    for x in da:
      x.wait()
    return carry
  lax.fori_loop(0, CHS_PER_TILE // 2, it_pay, 0)
  plsc.subcore_barrier()
  lax.fori_loop(0, EXPORT_CHS, export(out_x), 0)


def _sc_scatter(mm, pay, row):
  out_type = [jax.ShapeDtypeStruct((NC * NPAD, HID), jnp.float32),
              jax.ShapeDtypeStruct((NC * NPAD, HID), jnp.float32)]
  scratch = [
      pltpu.VMEM_SHARED((NPAD, HID), jnp.float32),
      pltpu.VMEM((CHS_PER_TILE, CS), jnp.int32),
      pltpu.VMEM((CS, HID), jnp.float32),
      pltpu.VMEM((CS, HID), jnp.float32),
      pltpu.VMEM((CS, 16), jnp.float32),
      pltpu.VMEM((CS, 16), jnp.float32),
      pltpu.SemaphoreType.DMA,
      pltpu.SemaphoreType.DMA,
      pltpu.SemaphoreType.DMA,
      pltpu.SemaphoreType.DMA,
      pltpu.SemaphoreType.DMA,
  ]
  fn = pl.kernel(_scatter_body, out_type=out_type, mesh=_mesh(),
                 scratch_types=scratch)
  return fn(mm, pay, row.reshape(-1, CS))


def _silu(v):
  return v * jax.nn.sigmoid(v)


def _rep(shape):
  nd = len(shape)
  return pl.BlockSpec(shape, lambda i: (0,) * nd)


def _node_pre_body(h, xa, wa, wb, ta, tb):
  hv = h[...]
  xav = xa[...]
  z = jnp.zeros((BN, 112), jnp.float32)
  ta[...] = jnp.concatenate(
      [jnp.dot(hv, wa[...], preferred_element_type=jnp.float32), xav, z], axis=1)
  tb[...] = jnp.concatenate(
      [jnp.dot(hv, wb[...], preferred_element_type=jnp.float32), xav, z], axis=1)


def _node_pre(h, xa, wa, wb):
  grid = NPAD // BN
  return pl.pallas_call(
      _node_pre_body,
      grid=(grid,),
      in_specs=[pl.BlockSpec((BN, HID), lambda i: (i, 0)),
                pl.BlockSpec((BN, 16), lambda i: (i, 0)),
                _rep((HID, HID)), _rep((HID, HID))],
      out_specs=[pl.BlockSpec((BN, 256), lambda i: (i, 0))] * 2,
      out_shape=[jax.ShapeDtypeStruct((NPAD, 256), jnp.float32)] * 2,
  )(h, xa, wa, wb)


def _edge_mlp_body(g1x, g2x, eat, wea, b1, wd, we2, b2,
                   watt, batt, wx1, bx1, wx2, bx2, mm_o, pay_o):
  lane = lax.broadcasted_iota(jnp.int32, (BE, 16), 1)
  g1v = g1x[...]
  g2v = g2x[...]
  diff = jnp.where(lane < 3, g1v[:, HID:HID + 16] - g2v[:, HID:HID + 16], 0.0)
  d2 = jnp.sum(diff * diff, axis=1, keepdims=True)
  pre = (g1v[:, 0:HID] + g2v[:, 0:HID] + d2 * wd[...]
         + jnp.dot(eat[...], wea[...], preferred_element_type=jnp.float32)
         + b1[...])
  m = _silu(pre)
  m = _silu(jnp.dot(m, we2[...], preferred_element_type=jnp.float32) + b2[...])
  att = jax.nn.sigmoid(
      jnp.dot(m, watt[...], preferred_element_type=jnp.float32) + batt[...])[:, 0:1]
  mm = m * att
  t = _silu(jnp.dot(mm, wx1[...], preferred_element_type=jnp.float32) + bx1[...])
  tx = (jnp.dot(t, wx2[...], preferred_element_type=jnp.float32) + bx2[...])[:, 0:1]
  mm_o[...] = mm
  pay_o[...] = diff * tx + (lane == 3).astype(jnp.float32)


def _edge_mlp(g1x, g2x, eat, w):
  ew = eat.shape[1]
  grid = EPAD // BE
  eb = pl.BlockSpec((BE, HID), lambda i: (i, 0))
  e256 = pl.BlockSpec((BE, 256), lambda i: (i, 0))
  e16 = pl.BlockSpec((BE, 16), lambda i: (i, 0))
  return pl.pallas_call(
      _edge_mlp_body,
      grid=(grid,),
      in_specs=[e256, e256, pl.BlockSpec((BE, ew), lambda i: (i, 0)),
                _rep((ew, HID)), _rep((1, HID)), _rep((1, HID)),
                _rep((HID, HID)), _rep((1, HID)),
                _rep((HID, 8)), _rep((1, 8)),
                _rep((HID, HID)), _rep((1, HID)),
                _rep((HID, 8)), _rep((1, 8))],
      out_specs=[eb, e16],
      out_shape=[jax.ShapeDtypeStruct((EPAD, HID), jnp.float32),
                 jax.ShapeDtypeStruct((EPAD, 16), jnp.float32)],
  )(g1x, g2x, eat, w["wea"], w["b1"], w["wd"], w["we2"], w["b2"],
    w["watt"], w["batt"], w["wx1"], w["bx1"], w["wx2"], w["bx2"])


def _node_upd_body(h, xa, p0, p1, q0, q1, wh1a, wh1b, bh1, wh2, bh2,
                   h_o, x_o):
  hv = h[...]
  agg = p0[...] + p1[...]
  u = _silu(jnp.dot(hv, wh1a[...], preferred_element_type=jnp.float32)
            + jnp.dot(agg, wh1b[...], preferred_element_type=jnp.float32)
            + bh1[...])
  h_o[...] = hv + jnp.dot(u, wh2[...], preferred_element_type=jnp.float32) + bh2[...]
  q = q0[...] + q1[...]
  cnt = q[:, 3:4]
  xv = xa[...]
  mask = xv[:, 4:5]
  upd = q[:, 0:16] * (mask / jnp.maximum(cnt, 1.0))
  lane = lax.broadcasted_iota(jnp.int32, (BN, 16), 1)
  x_o[...] = xv + jnp.where(lane < 3, upd, 0.0)


def _node_upd(h, xa, p0, p1, q0, q1, w):
  grid = NPAD // BN
  nb = pl.BlockSpec((BN, HID), lambda i: (i, 0))
  n16 = pl.BlockSpec((BN, 16), lambda i: (i, 0))
  return pl.pallas_call(
      _node_upd_body,
      grid=(grid,),
      in_specs=[nb, n16, nb, nb, nb, nb,
                _rep((HID, HID)), _rep((HID, HID)), _rep((1, HID)),
                _rep((HID, HID)), _rep((1, HID))],
      out_specs=[nb, n16],
      out_shape=[jax.ShapeDtypeStruct((NPAD, HID), jnp.float32),
                 jax.ShapeDtypeStruct((NPAD, 16), jnp.float32)],
  )(h, xa, p0, p1, q0, q1, w["wh1a"], w["wh1b"], w["bh1"], w["wh2"], w["bh2"])


def _edge_enc_body(g1x, g2x, emb, m1w, m1b, m2w, m2b, e_o):
  lane = lax.broadcasted_iota(jnp.int32, (BE, 16), 1)
  diff = jnp.where(lane < 3,
                   g1x[:, HID:HID + 16] - g2x[:, HID:HID + 16], 0.0)
  el = jnp.sqrt(jnp.sum(diff * diff, axis=1, keepdims=True) + 1e-12)
  d = jax.nn.relu(el * m1w[...] + m1b[...])
  d = jnp.dot(d, m2w[...], preferred_element_type=jnp.float32) + m2b[...]
  e_o[...] = d * emb[...]


def _edge_enc(g1x, g2x, emb, m1w, m1b, m2w, m2b):
  grid = EPAD // BE
  eb = pl.BlockSpec((BE, HID), lambda i: (i, 0))
  e256 = pl.BlockSpec((BE, 256), lambda i: (i, 0))
  return pl.pallas_call(
      _edge_enc_body,
      grid=(grid,),
      in_specs=[e256, e256, eb, _rep((1, HID)), _rep((1, HID)),
                _rep((HID, HID)), _rep((1, HID))],
      out_specs=eb,
      out_shape=jax.ShapeDtypeStruct((EPAD, HID), jnp.float32),
  )(g1x, g2x, emb, m1w, m1b, m2w, m2b)


def _edge_len_body(g1x, g2x, e_o):
  lane = lax.broadcasted_iota(jnp.int32, (BE, 16), 1)
  diff = jnp.where(lane < 3,
                   g1x[:, HID:HID + 16] - g2x[:, HID:HID + 16], 0.0)
  el = jnp.sqrt(jnp.sum(diff * diff, axis=1, keepdims=True) + 1e-12)
  e_o[...] = jnp.where(lane == 0, el, 0.0)


def _edge_len(g1x, g2x):
  grid = EPAD // BE
  e256 = pl.BlockSpec((BE, 256), lambda i: (i, 0))
  e16 = pl.BlockSpec((BE, 16), lambda i: (i, 0))
  return pl.pallas_call(
      _edge_len_body, grid=(grid,), in_specs=[e256, e256], out_specs=e16,
      out_shape=jax.ShapeDtypeStruct((EPAD, 16), jnp.float32),
  )(g1x, g2x)


def _init_body(ni, nemb, wnl, bnl, tsf, freqs, wt1, bt1, wt2, bt2,
               temb_tab, wtl, btl, h_o):
  ai = ni[:, 0:1]
  oh_a = (lax.broadcasted_iota(jnp.int32, (BN, 128), 1) == ai).astype(jnp.float32)
  sil_tab = _silu(nemb[...])
  nv = jnp.dot(jnp.dot(oh_a, sil_tab, preferred_element_type=jnp.float32),
               wnl[...], preferred_element_type=jnp.float32) + bnl[...]
  bi = ni[:, 1:2]
  oh_b = (lax.broadcasted_iota(jnp.int32, (BN, 64), 1) == bi).astype(jnp.float32)
  t0 = jnp.dot(oh_b, tsf[...], preferred_element_type=jnp.float32)[:, 0:1]
  ang = t0 * freqs[...]
  te = jnp.concatenate([jnp.sin(ang), jnp.cos(ang)], axis=1)
  te = _silu(jnp.dot(te, wt1[...], preferred_element_type=jnp.float32) + bt1[...])
  te = jnp.dot(te, wt2[...], preferred_element_type=jnp.float32) + bt2[...]
  mrows = jnp.dot(_silu(temb_tab[...]), wtl[...],
                  preferred_element_type=jnp.float32) + btl[...]
  tif = (ni[:, 2:3] == 1).astype(jnp.float32)
  memb = mrows[0:1, :] + tif * (mrows[1:2, :] - mrows[0:1, :])
  h_o[...] = jnp.concatenate([nv, te, memb], axis=1)


def _init_h(ni, p):
  grid = NPAD // BN
  return pl.pallas_call(
      _init_body,
      grid=(grid,),
      in_specs=[pl.BlockSpec((BN, 8), lambda i: (i, 0)),
                _rep((128, 64)), _rep((64, 64)), _rep((1, 64)),
                _rep((64, 8)), _rep((1, 16)),
                _rep((32, 32)), _rep((1, 32)), _rep((32, 32)), _rep((1, 32)),
                _rep((8, 32)), _rep((32, 32)), _rep((1, 32))],
      out_specs=pl.BlockSpec((BN, HID), lambda i: (i, 0)),
      out_shape=jax.ShapeDtypeStruct((NPAD, HID), jnp.float32),
  )(ni, p["nemb"], p["wnl"], p["bnl"], p["tsf"], p["freqs"],
    p["wt1"], p["bt1"], p["wt2"], p["bt2"], p["temb_tab"], p["wtl"], p["btl"])


def _prep_egcl(p):
  we1 = p["e1"]["w"]
  enf = we1.shape[0] - 2 * HID - 1
  wea = we1[2 * HID + 1:]
  if enf == 1:
    wea = jnp.pad(wea, ((0, 15), (0, 0)))
  return {
      "wa": we1[0:HID], "wb": we1[HID:2 * HID],
      "wd": we1[2 * HID:2 * HID + 1], "wea": wea,
      "b1": p["e1"]["b"][None, :],
      "we2": p["e2"]["w"], "b2": p["e2"]["b"][None, :],
      "watt": jnp.pad(p["att"]["w"], ((0, 0), (0, 7))),
      "batt": jnp.pad(p["att"]["b"][None, :], ((0, 0), (0, 7))),
      "wx1": p["x1"]["w"], "bx1": p["x1"]["b"][None, :],
      "wx2": jnp.pad(p["x2"]["w"], ((0, 0), (0, 7))),
      "bx2": jnp.pad(p["x2"]["b"][None, :], ((0, 0), (0, 7))),
      "wh1a": p["h1"]["w"][0:HID], "wh1b": p["h1"]["w"][HID:],
      "bh1": p["h1"]["b"][None, :],
      "wh2": p["h2"]["w"], "bh2": p["h2"]["b"][None, :],
  }


def _pad_edges(idx, fill):
  return jnp.concatenate(
      [idx.astype(jnp.int32),
       jnp.full((EPAD - N_EDGES,), fill, jnp.int32)])


def kernel(atom_type, pos, bond_index, bond_type, batch, graph_idx,
           time_step, template_mask, edge_index_a, params):
  del graph_idx
  row_b = _pad_edges(bond_index[0], DUMMY)
  col_b = _pad_edges(bond_index[1], 0)
  row_a = _pad_edges(edge_index_a[0], DUMMY)
  col_a = _pad_edges(edge_index_a[1], 0)
  typ = _pad_edges(bond_type, 0)

  maskf = template_mask.astype(jnp.float32)
  xa = jnp.zeros((NPAD, 16), jnp.float32)
  xa = xa.at[:N_NODES, 0:3].set(pos)
  xa = xa.at[:N_NODES, 4].set(maskf)

  ni = jnp.zeros((NPAD, 8), jnp.int32)
  ni = ni.at[:N_NODES, 0].set(atom_type.astype(jnp.int32))
  ni = ni.at[:N_NODES, 1].set(batch.astype(jnp.int32))
  ni = ni.at[:N_NODES, 2].set(template_mask.astype(jnp.int32))

  half = 16
  scale = math.log(10000.0) / (half - 1)
  freqs = jnp.exp(jnp.arange(half, dtype=jnp.float32) * -scale)[None, :]
  tsf = jnp.zeros((64, 8), jnp.float32).at[:, 0].set(
      time_step.astype(jnp.float32))

  ip = {
      "nemb": jnp.pad(params["node_emb"], ((0, 28), (0, 0))),
      "wnl": params["node_lin"]["w"], "bnl": params["node_lin"]["b"][None, :],
      "tsf": tsf, "freqs": freqs,
      "wt1": params["t1"]["w"], "bt1": params["t1"]["b"][None, :],
      "wt2": params["t2"]["w"], "bt2": params["t2"]["b"][None, :],
      "temb_tab": jnp.pad(params["tmpl_emb"], ((0, 6), (0, 0))),
      "wtl": params["tmpl_lin"]["w"], "btl": params["tmpl_lin"]["b"][None, :],
  }

  enc1 = [_prep_egcl(p) for p in params["enc1"]]
  encc = [_prep_egcl(p) for p in params["enc_cross"]]
  enc2 = [_prep_egcl(p) for p in params["enc2"]]

  ee1 = params["edge_enc"]
  ee2 = params["edge_enc2"]

  h = _init_h(ni, ip)

  emb1_pad = jnp.pad(ee1["emb"], ((0, 28), (0, 0)))
  emb2_pad = jnp.pad(ee2["emb"], ((0, 28), (0, 0)))
  emb_rows1, emb_rows2 = _sc_gather([emb1_pad, emb2_pad], [typ, typ])

  def gather_layer(h, xa, w, row, col):
    ta, tb = _node_pre(h, xa, w["wa"], w["wb"])
    return _sc_gather([ta, tb], [row, col])

  def finish_layer(h, xa, w, row, g1x, g2x, eat):
    mm, pay = _edge_mlp(g1x, g2x, eat, w)
    agg_h, agg_x = _sc_scatter(mm, pay, row)
    return _node_upd(h, xa, agg_h[:NPAD], agg_h[NPAD:],
                     agg_x[:NPAD], agg_x[NPAD:], w)

  e1buf = None
  for li, w in enumerate(enc1):
    g1x, g2x = gather_layer(h, xa, w, row_b, col_b)
    if li == 0:
      e1buf = _edge_enc(g1x, g2x, emb_rows1, ee1["m1"]["w"],
                        ee1["m1"]["b"][None, :], ee1["m2"]["w"],
                        ee1["m2"]["b"][None, :])
    h, xa = finish_layer(h, xa, w, row_b, g1x, g2x, e1buf)

  ela = None
  for li, w in enumerate(encc):
    g1x, g2x = gather_layer(h, xa, w, row_a, col_a)
    if li == 0:
      ela = _edge_len(g1x, g2x)
    h, xa = finish_layer(h, xa, w, row_a, g1x, g2x, ela)

  e2buf = None
  for li, w in enumerate(enc2):
    g1x, g2x = gather_layer(h, xa, w, row_b, col_b)
    if li == 0:
      e2buf = _edge_enc(g1x, g2x, emb_rows2, ee2["m1"]["w"],
                        ee2["m1"]["b"][None, :], ee2["m2"]["w"],
                        ee2["m2"]["b"][None, :])
    h, xa = finish_layer(h, xa, w, row_b, g1x, g2x, e2buf)

  return xa[:N_NODES, 0:3] - pos


# speedup vs baseline: 2.0468x; 1.0158x over previous
import functools
import math

import jax
import jax.numpy as jnp
from jax import lax
from jax.experimental import pallas as pl
from jax.experimental.pallas import tpu as pltpu
from jax.experimental.pallas import tpu_sc as plsc

N_NODES = 10000
N_EDGES = 160000
HID = 128

NC = 2
NS = 16
NW = NC * NS
CHUNK = 128

NPAD = 10240
DUMMY = N_NODES
EPAD = 163840
E_PER_TILE = EPAD // NW
CH_PER_TILE = E_PER_TILE // CHUNK
ROWS_PER_TILE = NPAD // NS
EXPORT_CH = ROWS_PER_TILE // CHUNK

BE = 1024
BN = 1024

_mesh = functools.partial(
    plsc.VectorSubcoreMesh, core_axis_name="c", subcore_axis_name="s",
    num_cores=NC, num_subcores=NS)


CHUNK_G = 80
CHG_PER_TILE = E_PER_TILE // CHUNK_G
GSLOTS = 2


def _gather_body(n_tasks, widths, *refs):
  tabs = refs[0:n_tasks]
  idxs = refs[n_tasks:2 * n_tasks]
  outs = refs[2 * n_tasks:3 * n_tasks]
  scr = refs[3 * n_tasks:]
  idx_all = scr[0:n_tasks]
  rb = scr[n_tasks:n_tasks + n_tasks * GSLOTS]
  sg = scr[n_tasks + n_tasks * GSLOTS:n_tasks + 2 * n_tasks * GSLOTS]
  sw = scr[n_tasks + 2 * n_tasks * GSLOTS:n_tasks + 3 * n_tasks * GSLOTS]
  si = scr[n_tasks + 3 * n_tasks * GSLOTS]
  cid = lax.axis_index("c")
  sid = lax.axis_index("s")
  wid = sid * NC + cid
  base = wid * E_PER_TILE
  base_c = wid * CHG_PER_TILE

  d = [pltpu.async_copy(idxs[k].at[pl.ds(base_c, CHG_PER_TILE)], idx_all[k], si)
       for k in range(n_tasks)]
  for x in d:
    x.wait()

  def drain_wb(s):
    for k in range(n_tasks):
      pltpu.make_async_copy(
          rb[k * GSLOTS + s], outs[k].at[pl.ds(base, CHUNK_G)],
          sw[k * GSLOTS + s]).wait()

  def it(c2, carry):
    gd = []
    for s in range(GSLOTS):
      c = GSLOTS * c2 + s

      @pl.when(c2 > 0)
      def _(s=s):
        drain_wb(s)

      for k in range(n_tasks):
        gd.append(pltpu.async_copy(
            tabs[k].at[idx_all[k].at[c]], rb[k * GSLOTS + s],
            sg[k * GSLOTS + s]))
    for s in range(GSLOTS):
      c = GSLOTS * c2 + s
      eb = base + c * CHUNK_G
      for k in range(n_tasks):
        gd[s * n_tasks + k].wait()
        pltpu.async_copy(
            rb[k * GSLOTS + s], outs[k].at[pl.ds(eb, CHUNK_G)],
            sw[k * GSLOTS + s])
    return carry

  lax.fori_loop(0, CHG_PER_TILE // GSLOTS, it, 0)
  for s in range(GSLOTS):
    drain_wb(s)


def _sc_gather(tables, indices):
  n = len(tables)
  widths = [t.shape[1] for t in tables]
  out_type = [jax.ShapeDtypeStruct((EPAD, w), jnp.float32) for w in widths]
  scratch = ([pltpu.VMEM((CHG_PER_TILE, CHUNK_G), jnp.int32) for _ in range(n)]
             + [pltpu.VMEM((CHUNK_G, w), jnp.float32)
                for w in widths for _ in range(GSLOTS)]
             + [pltpu.SemaphoreType.DMA for _ in range(2 * n * GSLOTS + 1)])
  fn = pl.kernel(
      functools.partial(_gather_body, n, tuple(widths)),
      out_type=out_type, mesh=_mesh(), scratch_types=scratch)
  idx2 = [i.reshape(-1, CHUNK_G) for i in indices]
  return fn(*tables, *idx2)


CS = 64
CHS_PER_TILE = E_PER_TILE // CS
EXPORT_CHS = ROWS_PER_TILE // CS


def _scatter_body(mm, pay, row2, out_h, out_x, acc, idx_all,
                  bm0, bm1, bp0, bp1, sl0, sl1, sa0, sa1, si):
  cid = lax.axis_index("c")
  sid = lax.axis_index("s")
  wid = sid * NC + cid
  base = wid * E_PER_TILE
  base_c = wid * CHS_PER_TILE
  bms = (bm0, bm1)
  bps = (bp0, bp1)
  sls = (sl0, sl1)
  sas = (sa0, sa1)

  pltpu.async_copy(row2.at[pl.ds(base_c, CHS_PER_TILE)], idx_all, si).wait()

  zero = jnp.zeros((16,), jnp.float32)

  def zinit(buf):
    def body(i, carry):
      for j in range(HID // 16):
        buf[i, pl.ds(j * 16, 16)] = zero
      return carry
    return body

  def zfill(k, carry):
    r = sid * ROWS_PER_TILE + k * CS
    pltpu.sync_copy(bm0, acc.at[pl.ds(r, CS)])
    return carry

  def export(out):
    def body(k, carry):
      r = sid * ROWS_PER_TILE + k * CS
      pltpu.sync_copy(acc.at[pl.ds(r, CS)], bm0)
      pltpu.sync_copy(bm0, out.at[pl.ds(cid * NPAD + r, CS)])
      return carry
    return body

  lax.fori_loop(0, CS, zinit(bm0), 0)
  lax.fori_loop(0, EXPORT_CHS, zfill, 0)
  plsc.subcore_barrier()

  def drain_add(s):
    pltpu.make_async_copy(bms[s], acc.at[pl.ds(0, CS)], sas[s]).wait()

  def it_mm(c2, carry):
    dl = []
    for s in range(2):
      c = 2 * c2 + s

      @pl.when(c2 > 0)
      def _(s=s):
        drain_add(s)

      dl.append(pltpu.async_copy(mm.at[pl.ds(base + c * CS, CS)],
                                 bms[s], sls[s]))
    for s in range(2):
      c = 2 * c2 + s
      dl[s].wait()
      pltpu.async_copy(bms[s], acc.at[idx_all.at[c]], sas[s], add=True)
    return carry
  lax.fori_loop(0, CHS_PER_TILE // 2, it_mm, 0)
  for s in range(2):
    drain_add(s)
  plsc.subcore_barrier()
  lax.fori_loop(0, EXPORT_CHS, export(out_h), 0)
  plsc.subcore_barrier()

  lax.fori_loop(0, CS, zinit(bm0), 0)
  lax.fori_loop(0, EXPORT_CHS, zfill, 0)
  plsc.subcore_barrier()
  lax.fori_loop(0, CS, zinit(bm0), 0)
  lax.fori_loop(0, CS, zinit(bm1), 0)

  def it_pay(c2, carry):
    dl = []
    for s in range(2):
      c = 2 * c2 + s
      dl.append(pltpu.async_copy(pay.at[pl.ds(base + c * CS, CS)],
                                 bps[s], sls[s]))
    for s in range(2):
      c = 2 * c2 + s
      dl[s].wait()

      @pl.when(c2 > 0)
      def _(s=s):
        drain_add(s)

      def fill(i, carry2, _bm=bms[s], _bp=bps[s]):
        _bm[i, pl.ds(0, 16)] = _bp[i]
        return carry2
      lax.fori_loop(0, CS, fill, 0)
      pltpu.async_copy(bms[s], acc.at[idx_all.at[c]], sas[s], add=True)
    return carry
  lax.fori_loop(0, CHS_PER_TILE // 2, it_pay, 0)
  for s in range(2):
    drain_add(s)
  plsc.subcore_barrier()
  lax.fori_loop(0, EXPORT_CHS, export(out_x), 0)


def _sc_scatter(mm, pay, row):
  out_type = [jax.ShapeDtypeStruct((NC * NPAD, HID), jnp.float32),
              jax.ShapeDtypeStruct((NC * NPAD, HID), jnp.float32)]
  scratch = [
      pltpu.VMEM_SHARED((NPAD, HID), jnp.float32),
      pltpu.VMEM((CHS_PER_TILE, CS), jnp.int32),
      pltpu.VMEM((CS, HID), jnp.float32),
      pltpu.VMEM((CS, HID), jnp.float32),
      pltpu.VMEM((CS, 16), jnp.float32),
      pltpu.VMEM((CS, 16), jnp.float32),
      pltpu.SemaphoreType.DMA,
      pltpu.SemaphoreType.DMA,
      pltpu.SemaphoreType.DMA,
      pltpu.SemaphoreType.DMA,
      pltpu.SemaphoreType.DMA,
  ]
  fn = pl.kernel(_scatter_body, out_type=out_type, mesh=_mesh(),
                 scratch_types=scratch)
  return fn(mm, pay, row.reshape(-1, CS))


def _silu(v):
  return v * jax.nn.sigmoid(v)


def _rep(shape):
  nd = len(shape)
  return pl.BlockSpec(shape, lambda i: (0,) * nd)


def _node_pre_body(h, xa, wa, wb, ta, tb):
  hv = h[...]
  xav = xa[...]
  z = jnp.zeros((BN, 112), jnp.float32)
  ta[...] = jnp.concatenate(
      [jnp.dot(hv, wa[...], preferred_element_type=jnp.float32), xav, z], axis=1)
  tb[...] = jnp.concatenate(
      [jnp.dot(hv, wb[...], preferred_element_type=jnp.float32), xav, z], axis=1)


def _node_pre(h, xa, wa, wb):
  grid = NPAD // BN
  return pl.pallas_call(
      _node_pre_body,
      grid=(grid,),
      in_specs=[pl.BlockSpec((BN, HID), lambda i: (i, 0)),
                pl.BlockSpec((BN, 16), lambda i: (i, 0)),
                _rep((HID, HID)), _rep((HID, HID))],
      out_specs=[pl.BlockSpec((BN, 256), lambda i: (i, 0))] * 2,
      out_shape=[jax.ShapeDtypeStruct((NPAD, 256), jnp.float32)] * 2,
  )(h, xa, wa, wb)


def _edge_mlp_body(g1x, g2x, eat, wea, b1, wd, we2, b2,
                   watt, batt, wx1, bx1, wx2, bx2, mm_o, pay_o):
  lane = lax.broadcasted_iota(jnp.int32, (BE, 16), 1)
  g1v = g1x[...]
  g2v = g2x[...]
  diff = jnp.where(lane < 3, g1v[:, HID:HID + 16] - g2v[:, HID:HID + 16], 0.0)
  d2 = jnp.sum(diff * diff, axis=1, keepdims=True)
  pre = (g1v[:, 0:HID] + g2v[:, 0:HID] + d2 * wd[...]
         + jnp.dot(eat[...], wea[...], preferred_element_type=jnp.float32)
         + b1[...])
  m = _silu(pre)
  m = _silu(jnp.dot(m, we2[...], preferred_element_type=jnp.float32) + b2[...])
  att = jax.nn.sigmoid(
      jnp.dot(m, watt[...], preferred_element_type=jnp.float32) + batt[...])[:, 0:1]
  mm = m * att
  t = _silu(jnp.dot(mm, wx1[...], preferred_element_type=jnp.float32) + bx1[...])
  tx = (jnp.dot(t, wx2[...], preferred_element_type=jnp.float32) + bx2[...])[:, 0:1]
  mm_o[...] = mm
  pay_o[...] = diff * tx + (lane == 3).astype(jnp.float32)


def _edge_mlp(g1x, g2x, eat, w):
  ew = eat.shape[1]
  grid = EPAD // BE
  eb = pl.BlockSpec((BE, HID), lambda i: (i, 0))
  e256 = pl.BlockSpec((BE, 256), lambda i: (i, 0))
  e16 = pl.BlockSpec((BE, 16), lambda i: (i, 0))
  return pl.pallas_call(
      _edge_mlp_body,
      grid=(grid,),
      in_specs=[e256, e256, pl.BlockSpec((BE, ew), lambda i: (i, 0)),
                _rep((ew, HID)), _rep((1, HID)), _rep((1, HID)),
                _rep((HID, HID)), _rep((1, HID)),
                _rep((HID, 8)), _rep((1, 8)),
                _rep((HID, HID)), _rep((1, HID)),
                _rep((HID, 8)), _rep((1, 8))],
      out_specs=[eb, e16],
      out_shape=[jax.ShapeDtypeStruct((EPAD, HID), jnp.float32),
                 jax.ShapeDtypeStruct((EPAD, 16), jnp.float32)],
  )(g1x, g2x, eat, w["wea"], w["b1"], w["wd"], w["we2"], w["b2"],
    w["watt"], w["batt"], w["wx1"], w["bx1"], w["wx2"], w["bx2"])


def _node_upd_body(h, xa, p0, p1, q0, q1, wh1a, wh1b, bh1, wh2, bh2,
                   h_o, x_o):
  hv = h[...]
  agg = p0[...] + p1[...]
  u = _silu(jnp.dot(hv, wh1a[...], preferred_element_type=jnp.float32)
            + jnp.dot(agg, wh1b[...], preferred_element_type=jnp.float32)
            + bh1[...])
  h_o[...] = hv + jnp.dot(u, wh2[...], preferred_element_type=jnp.float32) + bh2[...]
  q = q0[...] + q1[...]
  cnt = q[:, 3:4]
  xv = xa[...]
  mask = xv[:, 4:5]
  upd = q[:, 0:16] * (mask / jnp.maximum(cnt, 1.0))
  lane = lax.broadcasted_iota(jnp.int32, (BN, 16), 1)
  x_o[...] = xv + jnp.where(lane < 3, upd, 0.0)


def _node_upd(h, xa, p0, p1, q0, q1, w):
  grid = NPAD // BN
  nb = pl.BlockSpec((BN, HID), lambda i: (i, 0))
  n16 = pl.BlockSpec((BN, 16), lambda i: (i, 0))
  return pl.pallas_call(
      _node_upd_body,
      grid=(grid,),
      in_specs=[nb, n16, nb, nb, nb, nb,
                _rep((HID, HID)), _rep((HID, HID)), _rep((1, HID)),
                _rep((HID, HID)), _rep((1, HID))],
      out_specs=[nb, n16],
      out_shape=[jax.ShapeDtypeStruct((NPAD, HID), jnp.float32),
                 jax.ShapeDtypeStruct((NPAD, 16), jnp.float32)],
  )(h, xa, p0, p1, q0, q1, w["wh1a"], w["wh1b"], w["bh1"], w["wh2"], w["bh2"])


def _edge_enc_body(g1x, g2x, emb, m1w, m1b, m2w, m2b, e_o):
  lane = lax.broadcasted_iota(jnp.int32, (BE, 16), 1)
  diff = jnp.where(lane < 3,
                   g1x[:, HID:HID + 16] - g2x[:, HID:HID + 16], 0.0)
  el = jnp.sqrt(jnp.sum(diff * diff, axis=1, keepdims=True) + 1e-12)
  d = jax.nn.relu(el * m1w[...] + m1b[...])
  d = jnp.dot(d, m2w[...], preferred_element_type=jnp.float32) + m2b[...]
  e_o[...] = d * emb[...]


def _edge_enc(g1x, g2x, emb, m1w, m1b, m2w, m2b):
  grid = EPAD // BE
  eb = pl.BlockSpec((BE, HID), lambda i: (i, 0))
  e256 = pl.BlockSpec((BE, 256), lambda i: (i, 0))
  return pl.pallas_call(
      _edge_enc_body,
      grid=(grid,),
      in_specs=[e256, e256, eb, _rep((1, HID)), _rep((1, HID)),
                _rep((HID, HID)), _rep((1, HID))],
      out_specs=eb,
      out_shape=jax.ShapeDtypeStruct((EPAD, HID), jnp.float32),
  )(g1x, g2x, emb, m1w, m1b, m2w, m2b)


def _edge_len_body(g1x, g2x, e_o):
  lane = lax.broadcasted_iota(jnp.int32, (BE, 16), 1)
  diff = jnp.where(lane < 3,
                   g1x[:, HID:HID + 16] - g2x[:, HID:HID + 16], 0.0)
  el = jnp.sqrt(jnp.sum(diff * diff, axis=1, keepdims=True) + 1e-12)
  e_o[...] = jnp.where(lane == 0, el, 0.0)


def _edge_len(g1x, g2x):
  grid = EPAD // BE
  e256 = pl.BlockSpec((BE, 256), lambda i: (i, 0))
  e16 = pl.BlockSpec((BE, 16), lambda i: (i, 0))
  return pl.pallas_call(
      _edge_len_body, grid=(grid,), in_specs=[e256, e256], out_specs=e16,
      out_shape=jax.ShapeDtypeStruct((EPAD, 16), jnp.float32),
  )(g1x, g2x)


def _init_body(ni, nemb, wnl, bnl, tsf, freqs, wt1, bt1, wt2, bt2,
               temb_tab, wtl, btl, h_o):
  ai = ni[:, 0:1]
  oh_a = (lax.broadcasted_iota(jnp.int32, (BN, 128), 1) == ai).astype(jnp.float32)
  sil_tab = _silu(nemb[...])
  nv = jnp.dot(jnp.dot(oh_a, sil_tab, preferred_element_type=jnp.float32),
               wnl[...], preferred_element_type=jnp.float32) + bnl[...]
  bi = ni[:, 1:2]
  oh_b = (lax.broadcasted_iota(jnp.int32, (BN, 64), 1) == bi).astype(jnp.float32)
  t0 = jnp.dot(oh_b, tsf[...], preferred_element_type=jnp.float32)[:, 0:1]
  ang = t0 * freqs[...]
  te = jnp.concatenate([jnp.sin(ang), jnp.cos(ang)], axis=1)
  te = _silu(jnp.dot(te, wt1[...], preferred_element_type=jnp.float32) + bt1[...])
  te = jnp.dot(te, wt2[...], preferred_element_type=jnp.float32) + bt2[...]
  mrows = jnp.dot(_silu(temb_tab[...]), wtl[...],
                  preferred_element_type=jnp.float32) + btl[...]
  tif = (ni[:, 2:3] == 1).astype(jnp.float32)
  memb = mrows[0:1, :] + tif * (mrows[1:2, :] - mrows[0:1, :])
  h_o[...] = jnp.concatenate([nv, te, memb], axis=1)


def _init_h(ni, p):
  grid = NPAD // BN
  return pl.pallas_call(
      _init_body,
      grid=(grid,),
      in_specs=[pl.BlockSpec((BN, 8), lambda i: (i, 0)),
                _rep((128, 64)), _rep((64, 64)), _rep((1, 64)),
                _rep((64, 8)), _rep((1, 16)),
                _rep((32, 32)), _rep((1, 32)), _rep((32, 32)), _rep((1, 32)),
                _rep((8, 32)), _rep((32, 32)), _rep((1, 32))],
      out_specs=pl.BlockSpec((BN, HID), lambda i: (i, 0)),
      out_shape=jax.ShapeDtypeStruct((NPAD, HID), jnp.float32),
  )(ni, p["nemb"], p["wnl"], p["bnl"], p["tsf"], p["freqs"],
    p["wt1"], p["bt1"], p["wt2"], p["bt2"], p["temb_tab"], p["wtl"], p["btl"])


def _prep_egcl(p):
  we1 = p["e1"]["w"]
  enf = we1.shape[0] - 2 * HID - 1
  wea = we1[2 * HID + 1:]
  if enf == 1:
    wea = jnp.pad(wea, ((0, 15), (0, 0)))
  return {
      "wa": we1[0:HID], "wb": we1[HID:2 * HID],
      "wd": we1[2 * HID:2 * HID + 1], "wea": wea,
      "b1": p["e1"]["b"][None, :],
      "we2": p["e2"]["w"], "b2": p["e2"]["b"][None, :],
      "watt": jnp.pad(p["att"]["w"], ((0, 0), (0, 7))),
      "batt": jnp.pad(p["att"]["b"][None, :], ((0, 0), (0, 7))),
      "wx1": p["x1"]["w"], "bx1": p["x1"]["b"][None, :],
      "wx2": jnp.pad(p["x2"]["w"], ((0, 0), (0, 7))),
      "bx2": jnp.pad(p["x2"]["b"][None, :], ((0, 0), (0, 7))),
      "wh1a": p["h1"]["w"][0:HID], "wh1b": p["h1"]["w"][HID:],
      "bh1": p["h1"]["b"][None, :],
      "wh2": p["h2"]["w"], "bh2": p["h2"]["b"][None, :],
  }


def _pad_edges(idx, fill):
  return jnp.concatenate(
      [idx.astype(jnp.int32),
       jnp.full((EPAD - N_EDGES,), fill, jnp.int32)])


def kernel(atom_type, pos, bond_index, bond_type, batch, graph_idx,
           time_step, template_mask, edge_index_a, params):
  del graph_idx
  row_b = _pad_edges(bond_index[0], DUMMY)
  col_b = _pad_edges(bond_index[1], 0)
  row_a = _pad_edges(edge_index_a[0], DUMMY)
  col_a = _pad_edges(edge_index_a[1], 0)
  typ = _pad_edges(bond_type, 0)

  maskf = template_mask.astype(jnp.float32)
  xa = jnp.zeros((NPAD, 16), jnp.float32)
  xa = xa.at[:N_NODES, 0:3].set(pos)
  xa = xa.at[:N_NODES, 4].set(maskf)

  ni = jnp.zeros((NPAD, 8), jnp.int32)
  ni = ni.at[:N_NODES, 0].set(atom_type.astype(jnp.int32))
  ni = ni.at[:N_NODES, 1].set(batch.astype(jnp.int32))
  ni = ni.at[:N_NODES, 2].set(template_mask.astype(jnp.int32))

  half = 16
  scale = math.log(10000.0) / (half - 1)
  freqs = jnp.exp(jnp.arange(half, dtype=jnp.float32) * -scale)[None, :]
  tsf = jnp.zeros((64, 8), jnp.float32).at[:, 0].set(
      time_step.astype(jnp.float32))

  ip = {
      "nemb": jnp.pad(params["node_emb"], ((0, 28), (0, 0))),
      "wnl": params["node_lin"]["w"], "bnl": params["node_lin"]["b"][None, :],
      "tsf": tsf, "freqs": freqs,
      "wt1": params["t1"]["w"], "bt1": params["t1"]["b"][None, :],
      "wt2": params["t2"]["w"], "bt2": params["t2"]["b"][None, :],
      "temb_tab": jnp.pad(params["tmpl_emb"], ((0, 6), (0, 0))),
      "wtl": params["tmpl_lin"]["w"], "btl": params["tmpl_lin"]["b"][None, :],
  }

  enc1 = [_prep_egcl(p) for p in params["enc1"]]
  encc = [_prep_egcl(p) for p in params["enc_cross"]]
  enc2 = [_prep_egcl(p) for p in params["enc2"]]

  ee1 = params["edge_enc"]
  ee2 = params["edge_enc2"]

  h = _init_h(ni, ip)

  emb1_pad = jnp.pad(ee1["emb"], ((0, 28), (0, 0)))
  emb2_pad = jnp.pad(ee2["emb"], ((0, 28), (0, 0)))
  emb_rows1, emb_rows2 = _sc_gather([emb1_pad, emb2_pad], [typ, typ])

  def gather_layer(h, xa, w, row, col):
    ta, tb = _node_pre(h, xa, w["wa"], w["wb"])
    return _sc_gather([ta, tb], [row, col])

  def finish_layer(h, xa, w, row, g1x, g2x, eat):
    mm, pay = _edge_mlp(g1x, g2x, eat, w)
    agg_h, agg_x = _sc_scatter(mm, pay, row)
    return _node_upd(h, xa, agg_h[:NPAD], agg_h[NPAD:],
                     agg_x[:NPAD], agg_x[NPAD:], w)

  e1buf = None
  for li, w in enumerate(enc1):
    g1x, g2x = gather_layer(h, xa, w, row_b, col_b)
    if li == 0:
      e1buf = _edge_enc(g1x, g2x, emb_rows1, ee1["m1"]["w"],
                        ee1["m1"]["b"][None, :], ee1["m2"]["w"],
                        ee1["m2"]["b"][None, :])
    h, xa = finish_layer(h, xa, w, row_b, g1x, g2x, e1buf)

  ela = None
  for li, w in enumerate(encc):
    g1x, g2x = gather_layer(h, xa, w, row_a, col_a)
    if li == 0:
      ela = _edge_len(g1x, g2x)
    h, xa = finish_layer(h, xa, w, row_a, g1x, g2x, ela)

  e2buf = None
  for li, w in enumerate(enc2):
    g1x, g2x = gather_layer(h, xa, w, row_b, col_b)
    if li == 0:
      e2buf = _edge_enc(g1x, g2x, emb_rows2, ee2["m1"]["w"],
                        ee2["m1"]["b"][None, :], ee2["m2"]["w"],
                        ee2["m2"]["b"][None, :])
    h, xa = finish_layer(h, xa, w, row_b, g1x, g2x, e2buf)

  return xa[:N_NODES, 0:3] - pos
